# Initial kernel scaffold; baseline (speedup 1.0000x reference)
import jax
import jax.numpy as jnp
from jax.experimental import pallas as pl


def kernel(node_features, edge_index, edge_features, params):
    raise NotImplementedError("write your pallas kernel here")


# speedup vs baseline: 1.3416x; 1.3416x over previous
import functools

import jax
import jax.numpy as jnp
from jax import lax
from jax.experimental import pallas as pl
from jax.experimental.pallas import tpu as pltpu
from jax.experimental.pallas import tpu_sc as plsc

N_NODES = 10000
N_EDGES = 320000
D = 128

NC = 2
NS = 16
NW = NC * NS
EPW = N_EDGES // NW
CHUNK = 80
NCHUNK = EPW // CHUNK
ROWS_PER_TILE = 624
ROWS_REM = N_NODES - ROWS_PER_TILE * NS


def _ln(x, g, b):
    mu = jnp.mean(x, axis=-1, keepdims=True)
    var = jnp.mean((x - mu) ** 2, axis=-1, keepdims=True)
    return (x - mu) / jnp.sqrt(var + 1e-5) * g + b


def _dot(x, w):
    return jnp.dot(x, w, preferred_element_type=jnp.float32,
                   precision=jax.lax.Precision.HIGHEST)


def _mlp3_ln_body(x_ref, w0, b0, w1, b1, w2, b2, g, be, o_ref):
    h = jax.nn.relu(_dot(x_ref[...], w0[...]) + b0[...])
    h = jax.nn.relu(_dot(h, w1[...]) + b1[...])
    h = _dot(h, w2[...]) + b2[...]
    o_ref[...] = _ln(h, g[...], be[...])


def _edge_step_body(xs_ref, xr_ref, el_ref, ws, wr, we, b1, w1, bb1, w2, b2,
                    g, be, o_ref):
    el = el_ref[...]
    h = _dot(xs_ref[...], ws[...]) + _dot(xr_ref[...], wr[...]) \
        + _dot(el, we[...]) + b1[...]
    h = jax.nn.relu(h)
    h = jax.nn.relu(_dot(h, w1[...]) + bb1[...])
    h = _dot(h, w2[...]) + b2[...]
    o_ref[...] = el + _ln(h, g[...], be[...])


def _node_step_body(nl_ref, p0_ref, p1_ref, wa, wb, b0, w1, b1, w2, b2,
                    g, be, o_ref):
    nl = nl_ref[...]
    agg = p0_ref[...] + p1_ref[...]
    h = jax.nn.relu(_dot(nl, wa[...]) + _dot(agg, wb[...]) + b0[...])
    h = jax.nn.relu(_dot(h, w1[...]) + b1[...])
    h = _dot(h, w2[...]) + b2[...]
    o_ref[...] = nl + _ln(h, g[...], be[...])


def _dec_body(x_ref, w0, b0, w1, b1, w2, b2, o_ref):
    h = jax.nn.relu(_dot(x_ref[...], w0[...]) + b0[...])
    h = jax.nn.relu(_dot(h, w1[...]) + b1[...])
    o_ref[...] = _dot(h, w2[...]) + b2[...]


def _full_spec(shape):
    nd = len(shape)
    return pl.BlockSpec(shape, lambda i: (0,) * nd)


def _row_spec(bm, cols):
    return pl.BlockSpec((bm, cols), lambda i: (i, 0))


def _run_rows(body, x_list, w_list, out_cols, bm):
    rows = x_list[0].shape[0]
    grid = rows // bm
    in_specs = [_row_spec(bm, x.shape[1]) for x in x_list] \
        + [_full_spec(w.shape) for w in w_list]
    return pl.pallas_call(
        body,
        grid=(grid,),
        in_specs=in_specs,
        out_specs=_row_spec(bm, out_cols),
        out_shape=jax.ShapeDtypeStruct((rows, out_cols), jnp.float32),
    )(*x_list, *w_list)


_MESH = plsc.VectorSubcoreMesh(core_axis_name="c", subcore_axis_name="s")


@functools.partial(
    pl.kernel,
    out_type=[jax.ShapeDtypeStruct((N_EDGES, D), jnp.float32),
              jax.ShapeDtypeStruct((N_EDGES, D), jnp.float32)],
    mesh=_MESH,
    scratch_types=[
        pltpu.VMEM((CHUNK,), jnp.int32),
        pltpu.VMEM((CHUNK,), jnp.int32),
        pltpu.VMEM((CHUNK, D), jnp.float32),
        pltpu.VMEM((CHUNK, D), jnp.float32),
        pltpu.SemaphoreType.DMA,
    ],
)
def _sc_gather(nl_hbm, s_hbm, r_hbm, xs_hbm, xr_hbm,
               si_v, ri_v, rs_v, rr_v, sem):
    wid = lax.axis_index("s") * NC + lax.axis_index("c")
    base = wid * EPW

    def body(j, carry):
        off = pl.multiple_of(base + j * CHUNK, 8)
        pltpu.sync_copy(s_hbm.at[pl.ds(off, CHUNK)], si_v)
        pltpu.sync_copy(r_hbm.at[pl.ds(off, CHUNK)], ri_v)
        a = pltpu.async_copy(nl_hbm.at[si_v], rs_v, sem)
        b = pltpu.async_copy(nl_hbm.at[ri_v], rr_v, sem)
        a.wait()
        b.wait()
        pltpu.sync_copy(rs_v, xs_hbm.at[pl.ds(off, CHUNK)])
        pltpu.sync_copy(rr_v, xr_hbm.at[pl.ds(off, CHUNK)])
        return carry

    lax.fori_loop(0, NCHUNK, body, 0)


@functools.partial(
    pl.kernel,
    out_type=[jax.ShapeDtypeStruct((N_NODES, D), jnp.float32),
              jax.ShapeDtypeStruct((N_NODES, D), jnp.float32)],
    mesh=_MESH,
    scratch_types=[
        pltpu.VMEM((CHUNK,), jnp.int32),
        pltpu.VMEM((CHUNK, D), jnp.float32),
        pltpu.VMEM_SHARED((N_NODES, D), jnp.float32),
    ],
)
def _sc_scatter(ne_hbm, r_hbm, z_hbm, p0_hbm, p1_hbm, ri_v, rows_v, acc):
    cid = lax.axis_index("c")
    sid = lax.axis_index("s")
    wid = sid * NC + cid
    base = wid * EPW
    tile_row = sid * ROWS_PER_TILE

    pltpu.sync_copy(z_hbm.at[pl.ds(tile_row, ROWS_PER_TILE)],
                    acc.at[pl.ds(tile_row, ROWS_PER_TILE)])

    @pl.when(sid == NS - 1)
    def _():
        pltpu.sync_copy(z_hbm.at[pl.ds(ROWS_PER_TILE * NS, ROWS_REM)],
                        acc.at[pl.ds(ROWS_PER_TILE * NS, ROWS_REM)])

    plsc.subcore_barrier()

    def body(j, carry):
        off = pl.multiple_of(base + j * CHUNK, 8)
        pltpu.sync_copy(r_hbm.at[pl.ds(off, CHUNK)], ri_v)
        pltpu.sync_copy(ne_hbm.at[pl.ds(off, CHUNK)], rows_v)
        pltpu.sync_copy(rows_v, acc.at[ri_v], add=True)
        return carry

    lax.fori_loop(0, NCHUNK, body, 0)
    plsc.subcore_barrier()

    @pl.when(cid == 0)
    def _():
        pltpu.sync_copy(acc.at[pl.ds(tile_row, ROWS_PER_TILE)],
                        p0_hbm.at[pl.ds(tile_row, ROWS_PER_TILE)])

    @pl.when(cid == 1)
    def _():
        pltpu.sync_copy(acc.at[pl.ds(tile_row, ROWS_PER_TILE)],
                        p1_hbm.at[pl.ds(tile_row, ROWS_PER_TILE)])

    @pl.when((sid == NS - 1) & (cid == 0))
    def _():
        pltpu.sync_copy(acc.at[pl.ds(ROWS_PER_TILE * NS, ROWS_REM)],
                        p0_hbm.at[pl.ds(ROWS_PER_TILE * NS, ROWS_REM)])

    @pl.when((sid == NS - 1) & (cid == 1))
    def _():
        pltpu.sync_copy(acc.at[pl.ds(ROWS_PER_TILE * NS, ROWS_REM)],
                        p1_hbm.at[pl.ds(ROWS_PER_TILE * NS, ROWS_REM)])


def _prep_mlp(p, pad_in=None):
    ws = [w for w in p['W']]
    bs = [b.reshape(1, -1) for b in p['b']]
    if pad_in is not None:
        ws[0] = jnp.pad(ws[0], ((0, pad_in - ws[0].shape[0]), (0, 0)))
    out = []
    for w, b in zip(ws, bs):
        out += [w, b]
    if 'ln_scale' in p:
        out += [p['ln_scale'].reshape(1, -1), p['ln_bias'].reshape(1, -1)]
    return out


def kernel(node_features, edge_index, edge_features, params):
    senders = edge_index[0].astype(jnp.int32)
    receivers = edge_index[1].astype(jnp.int32)

    enc_n = _prep_mlp(params['enc_node'])
    enc_e = _prep_mlp(params['enc_edge'], pad_in=8)
    ef8 = jnp.pad(edge_features, ((0, 0), (0, 4)))

    nl = _run_rows(_mlp3_ln_body, [node_features], enc_n, D, 1000)
    el = _run_rows(_mlp3_ln_body, [ef8], enc_e, D, 2000)

    zeros_tab = jnp.zeros((N_NODES, D), jnp.float32)

    for blk in params['blocks']:
        pe = blk['edge']
        w1 = pe['W'][0]
        ew = [w1[:D], w1[D:2 * D], w1[2 * D:], pe['b'][0].reshape(1, -1),
              pe['W'][1], pe['b'][1].reshape(1, -1),
              pe['W'][2], pe['b'][2].reshape(1, -1),
              pe['ln_scale'].reshape(1, -1), pe['ln_bias'].reshape(1, -1)]
        pn = blk['node']
        nw1 = pn['W'][0]
        nwl = [nw1[:D], nw1[D:], pn['b'][0].reshape(1, -1),
               pn['W'][1], pn['b'][1].reshape(1, -1),
               pn['W'][2], pn['b'][2].reshape(1, -1),
               pn['ln_scale'].reshape(1, -1), pn['ln_bias'].reshape(1, -1)]

        xs, xr = _sc_gather(nl, senders, receivers)
        el = _run_rows(_edge_step_body, [xs, xr, el], ew, D, 2000)
        p0, p1 = _sc_scatter(el, receivers, zeros_tab)
        nl = _run_rows(_node_step_body, [nl, p0, p1], nwl, D, 1000)

    dec = _prep_mlp(params['dec'])
    dec[4] = jnp.pad(dec[4], ((0, 0), (0, 5)))
    dec[5] = jnp.pad(dec[5], ((0, 0), (0, 5)))
    out8 = _run_rows(_dec_body, [nl], dec, 8, 1000)
    return out8[:, :3]


# speedup vs baseline: 2.0583x; 1.5342x over previous
import functools

import jax
import jax.numpy as jnp
from jax import lax
from jax.experimental import pallas as pl
from jax.experimental.pallas import tpu as pltpu
from jax.experimental.pallas import tpu_sc as plsc

N_NODES = 10000
N_EDGES = 320000
D = 128

NC = 2
NS = 16
NW = NC * NS
EPW = N_EDGES // NW
CHUNK = 400
NCHUNK = EPW // CHUNK
CHUNK_S = 200
NCHUNK_S = EPW // CHUNK_S
ROWS_PER_TILE = 624
ROWS_REM = N_NODES - ROWS_PER_TILE * NS


def _ln(x, g, b):
    mu = jnp.mean(x, axis=-1, keepdims=True)
    var = jnp.mean((x - mu) ** 2, axis=-1, keepdims=True)
    return (x - mu) / jnp.sqrt(var + 1e-5) * g + b


def _dot(x, w):
    return jnp.dot(x, w, preferred_element_type=jnp.float32,
                   precision=jax.lax.Precision.HIGHEST)


def _enc_node_body(x_ref, w0, b0, w1, b1, w2, b2, g, be, ws, bs, wr,
                   o_ref, a_ref, t_ref):
    h = jax.nn.relu(_dot(x_ref[...], w0[...]) + b0[...])
    h = jax.nn.relu(_dot(h, w1[...]) + b1[...])
    h = _dot(h, w2[...]) + b2[...]
    nl = _ln(h, g[...], be[...])
    o_ref[...] = nl
    a_ref[...] = _dot(nl, ws[...]) + bs[...]
    t_ref[...] = _dot(nl, wr[...])


def _enc_edge_body(x_ref, w0, b0, w1, b1, w2, b2, g, be, o_ref):
    h = jax.nn.relu(_dot(x_ref[...], w0[...]) + b0[...])
    h = jax.nn.relu(_dot(h, w1[...]) + b1[...])
    h = _dot(h, w2[...]) + b2[...]
    o_ref[...] = _ln(h, g[...], be[...])


def _edge_step_body(u1_ref, el_ref, we, w1, bb1, w2, b2, g, be, o_ref):
    el = el_ref[...]
    h = jax.nn.relu(u1_ref[...] + _dot(el, we[...]))
    h = jax.nn.relu(_dot(h, w1[...]) + bb1[...])
    h = _dot(h, w2[...]) + b2[...]
    o_ref[...] = el + _ln(h, g[...], be[...])


def _node_step_body(nl_ref, p0_ref, p1_ref, wa, wb, b0, w1, b1, w2, b2,
                    g, be, ws, bs, wr, o_ref, a_ref, t_ref):
    nl = nl_ref[...]
    agg = p0_ref[...] + p1_ref[...]
    h = jax.nn.relu(_dot(nl, wa[...]) + _dot(agg, wb[...]) + b0[...])
    h = jax.nn.relu(_dot(h, w1[...]) + b1[...])
    h = _dot(h, w2[...]) + b2[...]
    nl = nl + _ln(h, g[...], be[...])
    o_ref[...] = nl
    a_ref[...] = _dot(nl, ws[...]) + bs[...]
    t_ref[...] = _dot(nl, wr[...])


def _node_last_body(nl_ref, p0_ref, p1_ref, wa, wb, b0, w1, b1, w2, b2,
                    g, be, o_ref):
    nl = nl_ref[...]
    agg = p0_ref[...] + p1_ref[...]
    h = jax.nn.relu(_dot(nl, wa[...]) + _dot(agg, wb[...]) + b0[...])
    h = jax.nn.relu(_dot(h, w1[...]) + b1[...])
    h = _dot(h, w2[...]) + b2[...]
    o_ref[...] = nl + _ln(h, g[...], be[...])


def _dec_body(x_ref, w0, b0, w1, b1, w2, b2, o_ref):
    h = jax.nn.relu(_dot(x_ref[...], w0[...]) + b0[...])
    h = jax.nn.relu(_dot(h, w1[...]) + b1[...])
    o_ref[...] = _dot(h, w2[...]) + b2[...]


def _full_spec(shape):
    nd = len(shape)
    return pl.BlockSpec(shape, lambda i: (0,) * nd)


def _row_spec(bm, cols):
    return pl.BlockSpec((bm, cols), lambda i: (i, 0))


def _run_rows(body, x_list, w_list, out_cols, bm, n_out=1):
    rows = x_list[0].shape[0]
    grid = rows // bm
    in_specs = [_row_spec(bm, x.shape[1]) for x in x_list] \
        + [_full_spec(w.shape) for w in w_list]
    if n_out == 1:
        out_specs = _row_spec(bm, out_cols)
        out_shape = jax.ShapeDtypeStruct((rows, out_cols), jnp.float32)
    else:
        out_specs = [_row_spec(bm, out_cols) for _ in range(n_out)]
        out_shape = [jax.ShapeDtypeStruct((rows, out_cols), jnp.float32)
                     for _ in range(n_out)]
    return pl.pallas_call(
        body,
        grid=(grid,),
        in_specs=in_specs,
        out_specs=out_specs,
        out_shape=out_shape,
    )(*x_list, *w_list)


_MESH = plsc.VectorSubcoreMesh(core_axis_name="c", subcore_axis_name="s")


@functools.partial(
    pl.kernel,
    out_type=jax.ShapeDtypeStruct((N_EDGES, D), jnp.float32),
    mesh=_MESH,
    scratch_types=[
        pltpu.VMEM((CHUNK,), jnp.int32),
        pltpu.VMEM((CHUNK,), jnp.int32),
        pltpu.VMEM((CHUNK, D), jnp.float32),
        pltpu.VMEM((CHUNK, D), jnp.float32),
        pltpu.SemaphoreType.DMA,
    ],
)
def _sc_gather(a_hbm, b_hbm, s_hbm, r_hbm, u1_hbm,
               si_v, ri_v, ra_v, rb_v, sem):
    wid = lax.axis_index("s") * NC + lax.axis_index("c")
    base = wid * EPW

    def body(j, carry):
        off = pl.multiple_of(base + j * CHUNK, 8)
        pltpu.sync_copy(s_hbm.at[pl.ds(off, CHUNK)], si_v)
        pltpu.sync_copy(r_hbm.at[pl.ds(off, CHUNK)], ri_v)
        da = pltpu.async_copy(a_hbm.at[si_v], ra_v, sem)
        db = pltpu.async_copy(b_hbm.at[ri_v], rb_v, sem)
        da.wait()
        db.wait()

        def add_row(e, c):
            for k in range(D // 16):
                sl = pl.ds(k * 16, 16)
                ra_v[e, sl] = ra_v[e, sl] + rb_v[e, sl]
            return c

        lax.fori_loop(0, CHUNK, add_row, 0)
        pltpu.sync_copy(ra_v, u1_hbm.at[pl.ds(off, CHUNK)])
        return carry

    lax.fori_loop(0, NCHUNK, body, 0)


@functools.partial(
    pl.kernel,
    out_type=[jax.ShapeDtypeStruct((N_NODES, D), jnp.float32),
              jax.ShapeDtypeStruct((N_NODES, D), jnp.float32)],
    mesh=_MESH,
    scratch_types=[
        pltpu.VMEM((CHUNK_S,), jnp.int32),
        pltpu.VMEM((CHUNK_S, D), jnp.float32),
        pltpu.VMEM_SHARED((N_NODES, D), jnp.float32),
    ],
)
def _sc_scatter(ne_hbm, r_hbm, z_hbm, p0_hbm, p1_hbm, ri_v, rows_v, acc):
    cid = lax.axis_index("c")
    sid = lax.axis_index("s")
    wid = sid * NC + cid
    base = wid * EPW
    tile_row = sid * ROWS_PER_TILE

    pltpu.sync_copy(z_hbm.at[pl.ds(tile_row, ROWS_PER_TILE)],
                    acc.at[pl.ds(tile_row, ROWS_PER_TILE)])

    @pl.when(sid == NS - 1)
    def _():
        pltpu.sync_copy(z_hbm.at[pl.ds(ROWS_PER_TILE * NS, ROWS_REM)],
                        acc.at[pl.ds(ROWS_PER_TILE * NS, ROWS_REM)])

    plsc.subcore_barrier()

    def body(j, carry):
        off = pl.multiple_of(base + j * CHUNK_S, 8)
        pltpu.sync_copy(r_hbm.at[pl.ds(off, CHUNK_S)], ri_v)
        pltpu.sync_copy(ne_hbm.at[pl.ds(off, CHUNK_S)], rows_v)
        pltpu.sync_copy(rows_v, acc.at[ri_v], add=True)
        return carry

    lax.fori_loop(0, NCHUNK_S, body, 0)
    plsc.subcore_barrier()

    @pl.when(cid == 0)
    def _():
        pltpu.sync_copy(acc.at[pl.ds(tile_row, ROWS_PER_TILE)],
                        p0_hbm.at[pl.ds(tile_row, ROWS_PER_TILE)])

    @pl.when(cid == 1)
    def _():
        pltpu.sync_copy(acc.at[pl.ds(tile_row, ROWS_PER_TILE)],
                        p1_hbm.at[pl.ds(tile_row, ROWS_PER_TILE)])

    @pl.when((sid == NS - 1) & (cid == 0))
    def _():
        pltpu.sync_copy(acc.at[pl.ds(ROWS_PER_TILE * NS, ROWS_REM)],
                        p0_hbm.at[pl.ds(ROWS_PER_TILE * NS, ROWS_REM)])

    @pl.when((sid == NS - 1) & (cid == 1))
    def _():
        pltpu.sync_copy(acc.at[pl.ds(ROWS_PER_TILE * NS, ROWS_REM)],
                        p1_hbm.at[pl.ds(ROWS_PER_TILE * NS, ROWS_REM)])


def _prep_mlp(p, pad_in=None):
    ws = [w for w in p['W']]
    bs = [b.reshape(1, -1) for b in p['b']]
    if pad_in is not None:
        ws[0] = jnp.pad(ws[0], ((0, pad_in - ws[0].shape[0]), (0, 0)))
    out = []
    for w, b in zip(ws, bs):
        out += [w, b]
    if 'ln_scale' in p:
        out += [p['ln_scale'].reshape(1, -1), p['ln_bias'].reshape(1, -1)]
    return out


def _edge_tables(blk):
    w1 = blk['edge']['W'][0]
    return w1[:D], blk['edge']['b'][0].reshape(1, -1), w1[D:2 * D]


def kernel(node_features, edge_index, edge_features, params):
    senders = edge_index[0].astype(jnp.int32)
    receivers = edge_index[1].astype(jnp.int32)
    blocks = params['blocks']

    enc_n = _prep_mlp(params['enc_node'])
    enc_e = _prep_mlp(params['enc_edge'], pad_in=8)
    ef8 = jnp.pad(edge_features, ((0, 0), (0, 4)))

    ws0, bs0, wr0 = _edge_tables(blocks[0])
    nl, ta, tb = _run_rows(_enc_node_body, [node_features],
                           enc_n + [ws0, bs0, wr0], D, 1000, n_out=3)
    el = _run_rows(_enc_edge_body, [ef8], enc_e, D, 2000)

    zeros_tab = jnp.zeros((N_NODES, D), jnp.float32)

    for s, blk in enumerate(blocks):
        pe = blk['edge']
        w1 = pe['W'][0]
        ew = [w1[2 * D:],
              pe['W'][1], pe['b'][1].reshape(1, -1),
              pe['W'][2], pe['b'][2].reshape(1, -1),
              pe['ln_scale'].reshape(1, -1), pe['ln_bias'].reshape(1, -1)]
        pn = blk['node']
        nw1 = pn['W'][0]
        nwl = [nw1[:D], nw1[D:], pn['b'][0].reshape(1, -1),
               pn['W'][1], pn['b'][1].reshape(1, -1),
               pn['W'][2], pn['b'][2].reshape(1, -1),
               pn['ln_scale'].reshape(1, -1), pn['ln_bias'].reshape(1, -1)]

        u1 = _sc_gather(ta, tb, senders, receivers)
        el = _run_rows(_edge_step_body, [u1, el], ew, D, 2000)
        p0, p1 = _sc_scatter(el, receivers, zeros_tab)
        if s + 1 < len(blocks):
            wsn, bsn, wrn = _edge_tables(blocks[s + 1])
            nl, ta, tb = _run_rows(_node_step_body, [nl, p0, p1],
                                   nwl + [wsn, bsn, wrn], D, 1000, n_out=3)
        else:
            nl = _run_rows(_node_last_body, [nl, p0, p1], nwl, D, 1000)

    dec = _prep_mlp(params['dec'])
    dec[4] = jnp.pad(dec[4], ((0, 0), (0, 5)))
    dec[5] = jnp.pad(dec[5], ((0, 0), (0, 5)))
    out8 = _run_rows(_dec_body, [nl], dec, 8, 1000)
    return out8[:, :3]


# speedup vs baseline: 2.1058x; 1.0231x over previous
import functools

import jax
import jax.numpy as jnp
from jax import lax
from jax.experimental import pallas as pl
from jax.experimental.pallas import tpu as pltpu
from jax.experimental.pallas import tpu_sc as plsc

N_NODES = 10000
N_EDGES = 320000
D = 128

NC = 2
NS = 16
NW = NC * NS
EPW = N_EDGES // NW
CHUNK = 200
NCHUNK = EPW // CHUNK
CHUNK_S = 80
NCHUNK_S = EPW // CHUNK_S
ROWS_PER_TILE = 624
ROWS_REM = N_NODES - ROWS_PER_TILE * NS


def _ln(x, g, b):
    mu = jnp.mean(x, axis=-1, keepdims=True)
    var = jnp.mean((x - mu) ** 2, axis=-1, keepdims=True)
    return (x - mu) / jnp.sqrt(var + 1e-5) * g + b


def _dot(x, w):
    return jnp.dot(x, w, preferred_element_type=jnp.float32,
                   precision=jax.lax.Precision.HIGHEST)


def _enc_node_body(x_ref, w0, b0, w1, b1, w2, b2, g, be, ws, bs, wr,
                   o_ref, a_ref, t_ref):
    h = jax.nn.relu(_dot(x_ref[...], w0[...]) + b0[...])
    h = jax.nn.relu(_dot(h, w1[...]) + b1[...])
    h = _dot(h, w2[...]) + b2[...]
    nl = _ln(h, g[...], be[...])
    o_ref[...] = nl
    a_ref[...] = _dot(nl, ws[...]) + bs[...]
    t_ref[...] = _dot(nl, wr[...])


def _enc_edge_body(x_ref, w0, b0, w1, b1, w2, b2, g, be, o_ref):
    h = jax.nn.relu(_dot(x_ref[...], w0[...]) + b0[...])
    h = jax.nn.relu(_dot(h, w1[...]) + b1[...])
    h = _dot(h, w2[...]) + b2[...]
    o_ref[...] = _ln(h, g[...], be[...])


def _edge_step_body(u1_ref, el_ref, we, w1, bb1, w2, b2, g, be, o_ref):
    el = el_ref[...]
    h = jax.nn.relu(u1_ref[...] + _dot(el, we[...]))
    h = jax.nn.relu(_dot(h, w1[...]) + bb1[...])
    h = _dot(h, w2[...]) + b2[...]
    o_ref[...] = el + _ln(h, g[...], be[...])


def _node_step_body(nl_ref, p0_ref, p1_ref, wa, wb, b0, w1, b1, w2, b2,
                    g, be, ws, bs, wr, o_ref, a_ref, t_ref):
    nl = nl_ref[...]
    agg = p0_ref[...] + p1_ref[...]
    h = jax.nn.relu(_dot(nl, wa[...]) + _dot(agg, wb[...]) + b0[...])
    h = jax.nn.relu(_dot(h, w1[...]) + b1[...])
    h = _dot(h, w2[...]) + b2[...]
    nl = nl + _ln(h, g[...], be[...])
    o_ref[...] = nl
    a_ref[...] = _dot(nl, ws[...]) + bs[...]
    t_ref[...] = _dot(nl, wr[...])


def _node_last_body(nl_ref, p0_ref, p1_ref, wa, wb, b0, w1, b1, w2, b2,
                    g, be, o_ref):
    nl = nl_ref[...]
    agg = p0_ref[...] + p1_ref[...]
    h = jax.nn.relu(_dot(nl, wa[...]) + _dot(agg, wb[...]) + b0[...])
    h = jax.nn.relu(_dot(h, w1[...]) + b1[...])
    h = _dot(h, w2[...]) + b2[...]
    o_ref[...] = nl + _ln(h, g[...], be[...])


def _dec_body(x_ref, w0, b0, w1, b1, w2, b2, o_ref):
    h = jax.nn.relu(_dot(x_ref[...], w0[...]) + b0[...])
    h = jax.nn.relu(_dot(h, w1[...]) + b1[...])
    o_ref[...] = _dot(h, w2[...]) + b2[...]


def _full_spec(shape):
    nd = len(shape)
    return pl.BlockSpec(shape, lambda i: (0,) * nd)


def _row_spec(bm, cols):
    return pl.BlockSpec((bm, cols), lambda i: (i, 0))


def _run_rows(body, x_list, w_list, out_cols, bm, n_out=1):
    rows = x_list[0].shape[0]
    grid = rows // bm
    in_specs = [_row_spec(bm, x.shape[1]) for x in x_list] \
        + [_full_spec(w.shape) for w in w_list]
    if n_out == 1:
        out_specs = _row_spec(bm, out_cols)
        out_shape = jax.ShapeDtypeStruct((rows, out_cols), jnp.float32)
    else:
        out_specs = [_row_spec(bm, out_cols) for _ in range(n_out)]
        out_shape = [jax.ShapeDtypeStruct((rows, out_cols), jnp.float32)
                     for _ in range(n_out)]
    return pl.pallas_call(
        body,
        grid=(grid,),
        in_specs=in_specs,
        out_specs=out_specs,
        out_shape=out_shape,
    )(*x_list, *w_list)


_MESH = plsc.VectorSubcoreMesh(core_axis_name="c", subcore_axis_name="s")


@functools.partial(
    pl.kernel,
    out_type=jax.ShapeDtypeStruct((N_EDGES, D), jnp.float32),
    mesh=_MESH,
    scratch_types=[
        [pltpu.VMEM((CHUNK,), jnp.int32) for _ in range(2)],
        [pltpu.VMEM((CHUNK,), jnp.int32) for _ in range(2)],
        [pltpu.VMEM((CHUNK, D), jnp.float32) for _ in range(2)],
        [pltpu.VMEM((CHUNK, D), jnp.float32) for _ in range(2)],
        [pltpu.SemaphoreType.DMA for _ in range(2)],
        [pltpu.SemaphoreType.DMA for _ in range(2)],
        [pltpu.SemaphoreType.DMA for _ in range(2)],
    ],
)
def _sc_gather(a_hbm, b_hbm, s_hbm, r_hbm, u1_hbm,
               si, ri, ra, rb, semi, semg, sems):
    wid = lax.axis_index("s") * NC + lax.axis_index("c")
    base = wid * EPW

    def issue_idx(c, t):
        off = pl.multiple_of(base + c * CHUNK, 8)
        pltpu.async_copy(s_hbm.at[pl.ds(off, CHUNK)], si[t], semi[t])
        pltpu.async_copy(r_hbm.at[pl.ds(off, CHUNK)], ri[t], semi[t])

    def wait_idx(t):
        pltpu.make_async_copy(s_hbm.at[pl.ds(0, CHUNK)], si[t], semi[t]).wait()
        pltpu.make_async_copy(r_hbm.at[pl.ds(0, CHUNK)], ri[t], semi[t]).wait()

    def issue_gather(t):
        pltpu.async_copy(a_hbm.at[si[t]], ra[t], semg[t])
        pltpu.async_copy(b_hbm.at[ri[t]], rb[t], semg[t])

    def wait_gather(t):
        pltpu.make_async_copy(u1_hbm.at[pl.ds(0, CHUNK)], ra[t], semg[t]).wait()
        pltpu.make_async_copy(u1_hbm.at[pl.ds(0, CHUNK)], rb[t], semg[t]).wait()

    def issue_store(c, t):
        off = pl.multiple_of(base + c * CHUNK, 8)
        pltpu.async_copy(ra[t], u1_hbm.at[pl.ds(off, CHUNK)], sems[t])

    def wait_store(t):
        pltpu.make_async_copy(ra[t], u1_hbm.at[pl.ds(0, CHUNK)], sems[t]).wait()

    def add(t):
        def add_row(e, c):
            for k in range(D // 16):
                sl = pl.ds(k * 16, 16)
                ra[t][e, sl] = ra[t][e, sl] + rb[t][e, sl]
            return c
        lax.fori_loop(0, CHUNK, add_row, 0)

    issue_idx(0, 0)
    issue_idx(1, 1)
    wait_idx(0)
    issue_gather(0)
    wait_idx(1)
    issue_gather(1)

    def body(k, carry):
        c = 2 * k
        wait_gather(0)
        issue_idx(c + 2, 0)
        add(0)
        issue_store(c, 0)
        wait_gather(1)
        issue_idx(c + 3, 1)
        wait_store(0)
        wait_idx(0)
        issue_gather(0)
        add(1)
        issue_store(c + 1, 1)
        wait_store(1)
        wait_idx(1)
        issue_gather(1)
        return carry

    lax.fori_loop(0, NCHUNK // 2 - 1, body, 0)

    wait_gather(0)
    add(0)
    issue_store(NCHUNK - 2, 0)
    wait_gather(1)
    add(1)
    issue_store(NCHUNK - 1, 1)
    wait_store(0)
    wait_store(1)


@functools.partial(
    pl.kernel,
    out_type=[jax.ShapeDtypeStruct((N_NODES, D), jnp.float32),
              jax.ShapeDtypeStruct((N_NODES, D), jnp.float32)],
    mesh=_MESH,
    scratch_types=[
        [pltpu.VMEM((CHUNK_S,), jnp.int32) for _ in range(2)],
        [pltpu.VMEM((CHUNK_S, D), jnp.float32) for _ in range(2)],
        pltpu.VMEM_SHARED((N_NODES, D), jnp.float32),
        [pltpu.SemaphoreType.DMA for _ in range(2)],
    ],
)
def _sc_scatter(ne_hbm, r_hbm, z_hbm, p0_hbm, p1_hbm, ri, rows, acc, seml):
    cid = lax.axis_index("c")
    sid = lax.axis_index("s")
    wid = sid * NC + cid
    base = wid * EPW
    tile_row = sid * ROWS_PER_TILE

    pltpu.sync_copy(z_hbm.at[pl.ds(tile_row, ROWS_PER_TILE)],
                    acc.at[pl.ds(tile_row, ROWS_PER_TILE)])

    @pl.when(sid == NS - 1)
    def _():
        pltpu.sync_copy(z_hbm.at[pl.ds(ROWS_PER_TILE * NS, ROWS_REM)],
                        acc.at[pl.ds(ROWS_PER_TILE * NS, ROWS_REM)])

    plsc.subcore_barrier()

    def issue_loads(c, t):
        off = pl.multiple_of(base + c * CHUNK_S, 8)
        pltpu.async_copy(r_hbm.at[pl.ds(off, CHUNK_S)], ri[t], seml[t])
        pltpu.async_copy(ne_hbm.at[pl.ds(off, CHUNK_S)], rows[t], seml[t])

    def wait_loads(t):
        pltpu.make_async_copy(r_hbm.at[pl.ds(0, CHUNK_S)], ri[t],
                              seml[t]).wait()
        pltpu.make_async_copy(ne_hbm.at[pl.ds(0, CHUNK_S)], rows[t],
                              seml[t]).wait()

    def body(j, carry):
        off = pl.multiple_of(base + j * CHUNK_S, 8)
        pltpu.sync_copy(r_hbm.at[pl.ds(off, CHUNK_S)], ri[0])
        pltpu.sync_copy(ne_hbm.at[pl.ds(off, CHUNK_S)], rows[0])
        pltpu.sync_copy(rows[0], acc.at[ri[0]], add=True)
        return carry

    lax.fori_loop(0, NCHUNK_S, body, 0)
    plsc.subcore_barrier()

    @pl.when(cid == 0)
    def _():
        pltpu.sync_copy(acc.at[pl.ds(tile_row, ROWS_PER_TILE)],
                        p0_hbm.at[pl.ds(tile_row, ROWS_PER_TILE)])

    @pl.when(cid == 1)
    def _():
        pltpu.sync_copy(acc.at[pl.ds(tile_row, ROWS_PER_TILE)],
                        p1_hbm.at[pl.ds(tile_row, ROWS_PER_TILE)])

    @pl.when((sid == NS - 1) & (cid == 0))
    def _():
        pltpu.sync_copy(acc.at[pl.ds(ROWS_PER_TILE * NS, ROWS_REM)],
                        p0_hbm.at[pl.ds(ROWS_PER_TILE * NS, ROWS_REM)])

    @pl.when((sid == NS - 1) & (cid == 1))
    def _():
        pltpu.sync_copy(acc.at[pl.ds(ROWS_PER_TILE * NS, ROWS_REM)],
                        p1_hbm.at[pl.ds(ROWS_PER_TILE * NS, ROWS_REM)])


def _prep_mlp(p, pad_in=None):
    ws = [w for w in p['W']]
    bs = [b.reshape(1, -1) for b in p['b']]
    if pad_in is not None:
        ws[0] = jnp.pad(ws[0], ((0, pad_in - ws[0].shape[0]), (0, 0)))
    out = []
    for w, b in zip(ws, bs):
        out += [w, b]
    if 'ln_scale' in p:
        out += [p['ln_scale'].reshape(1, -1), p['ln_bias'].reshape(1, -1)]
    return out


def _edge_tables(blk):
    w1 = blk['edge']['W'][0]
    return w1[:D], blk['edge']['b'][0].reshape(1, -1), w1[D:2 * D]


def kernel(node_features, edge_index, edge_features, params):
    senders = edge_index[0].astype(jnp.int32)
    receivers = edge_index[1].astype(jnp.int32)
    blocks = params['blocks']

    enc_n = _prep_mlp(params['enc_node'])
    enc_e = _prep_mlp(params['enc_edge'], pad_in=8)
    ef8 = jnp.pad(edge_features, ((0, 0), (0, 4)))

    ws0, bs0, wr0 = _edge_tables(blocks[0])
    nl, ta, tb = _run_rows(_enc_node_body, [node_features],
                           enc_n + [ws0, bs0, wr0], D, 1000, n_out=3)
    el = _run_rows(_enc_edge_body, [ef8], enc_e, D, 2000)

    zeros_tab = jnp.zeros((N_NODES, D), jnp.float32)

    for s, blk in enumerate(blocks):
        pe = blk['edge']
        w1 = pe['W'][0]
        ew = [w1[2 * D:],
              pe['W'][1], pe['b'][1].reshape(1, -1),
              pe['W'][2], pe['b'][2].reshape(1, -1),
              pe['ln_scale'].reshape(1, -1), pe['ln_bias'].reshape(1, -1)]
        pn = blk['node']
        nw1 = pn['W'][0]
        nwl = [nw1[:D], nw1[D:], pn['b'][0].reshape(1, -1),
               pn['W'][1], pn['b'][1].reshape(1, -1),
               pn['W'][2], pn['b'][2].reshape(1, -1),
               pn['ln_scale'].reshape(1, -1), pn['ln_bias'].reshape(1, -1)]

        u1 = _sc_gather(ta, tb, senders, receivers)
        el = _run_rows(_edge_step_body, [u1, el], ew, D, 2000)
        p0, p1 = _sc_scatter(el, receivers, zeros_tab)
        if s + 1 < len(blocks):
            wsn, bsn, wrn = _edge_tables(blocks[s + 1])
            nl, ta, tb = _run_rows(_node_step_body, [nl, p0, p1],
                                   nwl + [wsn, bsn, wrn], D, 1000, n_out=3)
        else:
            nl = _run_rows(_node_last_body, [nl, p0, p1], nwl, D, 1000)

    dec = _prep_mlp(params['dec'])
    dec[4] = jnp.pad(dec[4], ((0, 0), (0, 5)))
    dec[5] = jnp.pad(dec[5], ((0, 0), (0, 5)))
    out8 = _run_rows(_dec_body, [nl], dec, 8, 1000)
    return out8[:, :3]


# speedup vs baseline: 2.2717x; 1.0788x over previous
import functools

import jax
import jax.numpy as jnp
from jax import lax
from jax.experimental import pallas as pl
from jax.experimental.pallas import tpu as pltpu
from jax.experimental.pallas import tpu_sc as plsc

N_NODES = 10000
N_EDGES = 320000
D = 128

NC = 2
NS = 16
NW = NC * NS
EPW = N_EDGES // NW
CHUNK = 200
NCHUNK = EPW // CHUNK
CHUNK_S = 80
NCHUNK_S = EPW // CHUNK_S
ROWS_PER_TILE = 624
ROWS_REM = N_NODES - ROWS_PER_TILE * NS


def _ln(x, g, b):
    mu = jnp.mean(x, axis=-1, keepdims=True)
    var = jnp.mean((x - mu) ** 2, axis=-1, keepdims=True)
    return (x - mu) / jnp.sqrt(var + 1e-5) * g + b


def _dot(x, w):
    return jnp.dot(x, w, preferred_element_type=jnp.float32,
                   precision=jax.lax.Precision.HIGHEST)


def _enc_node_body(x_ref, w0, b0, w1, b1, w2, b2, g, be, ws, bs, wr,
                   o_ref, a_ref, t_ref):
    h = jax.nn.relu(_dot(x_ref[...], w0[...]) + b0[...])
    h = jax.nn.relu(_dot(h, w1[...]) + b1[...])
    h = _dot(h, w2[...]) + b2[...]
    nl = _ln(h, g[...], be[...])
    o_ref[...] = nl
    a_ref[...] = _dot(nl, ws[...]) + bs[...]
    t_ref[...] = _dot(nl, wr[...])


def _enc_edge_body(x_ref, w0, b0, w1, b1, w2, b2, g, be, o_ref):
    h = jax.nn.relu(_dot(x_ref[...], w0[...]) + b0[...])
    h = jax.nn.relu(_dot(h, w1[...]) + b1[...])
    h = _dot(h, w2[...]) + b2[...]
    o_ref[...] = _ln(h, g[...], be[...])


def _edge_step_body(u1_ref, el_ref, we, w1, bb1, w2, b2, g, be, o_ref):
    el = el_ref[...]
    h = jax.nn.relu(u1_ref[...] + _dot(el, we[...]))
    h = jax.nn.relu(_dot(h, w1[...]) + bb1[...])
    h = _dot(h, w2[...]) + b2[...]
    o_ref[...] = el + _ln(h, g[...], be[...])


def _node_step_body(nl_ref, p0_ref, p1_ref, wa, wb, b0, w1, b1, w2, b2,
                    g, be, ws, bs, wr, o_ref, a_ref, t_ref):
    nl = nl_ref[...]
    agg = p0_ref[...] + p1_ref[...]
    h = jax.nn.relu(_dot(nl, wa[...]) + _dot(agg, wb[...]) + b0[...])
    h = jax.nn.relu(_dot(h, w1[...]) + b1[...])
    h = _dot(h, w2[...]) + b2[...]
    nl = nl + _ln(h, g[...], be[...])
    o_ref[...] = nl
    a_ref[...] = _dot(nl, ws[...]) + bs[...]
    t_ref[...] = _dot(nl, wr[...])


def _node_last_body(nl_ref, p0_ref, p1_ref, wa, wb, b0, w1, b1, w2, b2,
                    g, be, o_ref):
    nl = nl_ref[...]
    agg = p0_ref[...] + p1_ref[...]
    h = jax.nn.relu(_dot(nl, wa[...]) + _dot(agg, wb[...]) + b0[...])
    h = jax.nn.relu(_dot(h, w1[...]) + b1[...])
    h = _dot(h, w2[...]) + b2[...]
    o_ref[...] = nl + _ln(h, g[...], be[...])


def _dec_body(x_ref, w0, b0, w1, b1, w2, b2, o_ref):
    h = jax.nn.relu(_dot(x_ref[...], w0[...]) + b0[...])
    h = jax.nn.relu(_dot(h, w1[...]) + b1[...])
    o_ref[...] = _dot(h, w2[...]) + b2[...]


def _full_spec(shape):
    nd = len(shape)
    return pl.BlockSpec(shape, lambda i: (0,) * nd)


def _row_spec(bm, cols):
    return pl.BlockSpec((bm, cols), lambda i: (i, 0))


def _run_rows(body, x_list, w_list, out_cols, bm, n_out=1):
    rows = x_list[0].shape[0]
    grid = rows // bm
    in_specs = [_row_spec(bm, x.shape[1]) for x in x_list] \
        + [_full_spec(w.shape) for w in w_list]
    if n_out == 1:
        out_specs = _row_spec(bm, out_cols)
        out_shape = jax.ShapeDtypeStruct((rows, out_cols), jnp.float32)
    else:
        out_specs = [_row_spec(bm, out_cols) for _ in range(n_out)]
        out_shape = [jax.ShapeDtypeStruct((rows, out_cols), jnp.float32)
                     for _ in range(n_out)]
    return pl.pallas_call(
        body,
        grid=(grid,),
        in_specs=in_specs,
        out_specs=out_specs,
        out_shape=out_shape,
    )(*x_list, *w_list)


_MESH = plsc.VectorSubcoreMesh(core_axis_name="c", subcore_axis_name="s")


@functools.partial(
    pl.kernel,
    out_type=jax.ShapeDtypeStruct((N_EDGES, D), jnp.float32),
    mesh=_MESH,
    scratch_types=[
        [pltpu.VMEM((CHUNK,), jnp.int32) for _ in range(2)],
        [pltpu.VMEM((CHUNK,), jnp.int32) for _ in range(2)],
        [pltpu.VMEM((CHUNK, D), jnp.float32) for _ in range(2)],
        [pltpu.VMEM((CHUNK, D), jnp.float32) for _ in range(2)],
        [pltpu.SemaphoreType.DMA for _ in range(2)],
        [pltpu.SemaphoreType.DMA for _ in range(2)],
        [pltpu.SemaphoreType.DMA for _ in range(2)],
    ],
)
def _sc_gather(a_hbm, b_hbm, s_hbm, r_hbm, u1_hbm,
               si, ri, ra, rb, semi, semg, sems):
    wid = lax.axis_index("s") * NC + lax.axis_index("c")
    base = wid * EPW

    def issue_idx(c, t):
        off = pl.multiple_of(base + c * CHUNK, 8)
        pltpu.async_copy(s_hbm.at[pl.ds(off, CHUNK)], si[t], semi[t])
        pltpu.async_copy(r_hbm.at[pl.ds(off, CHUNK)], ri[t], semi[t])

    def wait_idx(t):
        pltpu.make_async_copy(s_hbm.at[pl.ds(0, CHUNK)], si[t], semi[t]).wait()
        pltpu.make_async_copy(r_hbm.at[pl.ds(0, CHUNK)], ri[t], semi[t]).wait()

    def issue_gather(t):
        pltpu.async_copy(a_hbm.at[si[t]], ra[t], semg[t])
        pltpu.async_copy(b_hbm.at[ri[t]], rb[t], semg[t])

    def wait_gather(t):
        pltpu.make_async_copy(u1_hbm.at[pl.ds(0, CHUNK)], ra[t], semg[t]).wait()
        pltpu.make_async_copy(u1_hbm.at[pl.ds(0, CHUNK)], rb[t], semg[t]).wait()

    def issue_store(c, t):
        off = pl.multiple_of(base + c * CHUNK, 8)
        pltpu.async_copy(ra[t], u1_hbm.at[pl.ds(off, CHUNK)], sems[t])

    def wait_store(t):
        pltpu.make_async_copy(ra[t], u1_hbm.at[pl.ds(0, CHUNK)], sems[t]).wait()

    def add(t):
        def add_row(e, c):
            for k in range(D // 16):
                sl = pl.ds(k * 16, 16)
                ra[t][e, sl] = ra[t][e, sl] + rb[t][e, sl]
            return c
        lax.fori_loop(0, CHUNK, add_row, 0)

    issue_idx(0, 0)
    issue_idx(1, 1)
    wait_idx(0)
    issue_gather(0)
    wait_idx(1)
    issue_gather(1)

    def body(k, carry):
        c = 2 * k
        wait_gather(0)
        issue_idx(c + 2, 0)
        add(0)
        issue_store(c, 0)
        wait_gather(1)
        issue_idx(c + 3, 1)
        wait_store(0)
        wait_idx(0)
        issue_gather(0)
        add(1)
        issue_store(c + 1, 1)
        wait_store(1)
        wait_idx(1)
        issue_gather(1)
        return carry

    lax.fori_loop(0, NCHUNK // 2 - 1, body, 0)

    wait_gather(0)
    add(0)
    issue_store(NCHUNK - 2, 0)
    wait_gather(1)
    add(1)
    issue_store(NCHUNK - 1, 1)
    wait_store(0)
    wait_store(1)


@functools.partial(
    pl.kernel,
    out_type=[jax.ShapeDtypeStruct((N_NODES, D), jnp.float32),
              jax.ShapeDtypeStruct((N_NODES, D), jnp.float32)],
    mesh=_MESH,
    scratch_types=[
        [pltpu.VMEM((CHUNK_S,), jnp.int32) for _ in range(2)],
        [pltpu.VMEM((CHUNK_S, D), jnp.float32) for _ in range(2)],
        pltpu.VMEM_SHARED((N_NODES, D), jnp.float32),
        [pltpu.SemaphoreType.DMA for _ in range(2)],
    ],
)
def _sc_scatter(ne_hbm, r_hbm, z_hbm, p0_hbm, p1_hbm, ri, rows, acc, seml):
    cid = lax.axis_index("c")
    sid = lax.axis_index("s")
    wid = sid * NC + cid
    base = wid * EPW
    tile_row = sid * ROWS_PER_TILE

    pltpu.sync_copy(z_hbm.at[pl.ds(tile_row, ROWS_PER_TILE)],
                    acc.at[pl.ds(tile_row, ROWS_PER_TILE)])

    @pl.when(sid == NS - 1)
    def _():
        pltpu.sync_copy(z_hbm.at[pl.ds(ROWS_PER_TILE * NS, ROWS_REM)],
                        acc.at[pl.ds(ROWS_PER_TILE * NS, ROWS_REM)])

    plsc.subcore_barrier()

    def issue_loads(c, t):
        off = pl.multiple_of(base + c * CHUNK_S, 8)
        pltpu.async_copy(r_hbm.at[pl.ds(off, CHUNK_S)], ri[t], seml[t])
        pltpu.async_copy(ne_hbm.at[pl.ds(off, CHUNK_S)], rows[t], seml[t])

    def wait_loads(t):
        pltpu.make_async_copy(r_hbm.at[pl.ds(0, CHUNK_S)], ri[t],
                              seml[t]).wait()
        pltpu.make_async_copy(ne_hbm.at[pl.ds(0, CHUNK_S)], rows[t],
                              seml[t]).wait()

    def issue_loads(c, t):
        off = pl.multiple_of(base + c * CHUNK_S, 8)
        di = pltpu.async_copy(r_hbm.at[pl.ds(off, CHUNK_S)], ri[t], seml[t])
        dr = pltpu.async_copy(ne_hbm.at[pl.ds(off, CHUNK_S)], rows[t],
                              seml[t])
        return di, dr

    def body(k, carry):
        c = 2 * k
        d0i, d0r = issue_loads(c, 0)
        d1i, d1r = issue_loads(c + 1, 1)
        d0i.wait()
        d0r.wait()
        pltpu.sync_copy(rows[0], acc.at[ri[0]], add=True)
        d1i.wait()
        d1r.wait()
        pltpu.sync_copy(rows[1], acc.at[ri[1]], add=True)
        return carry

    lax.fori_loop(0, NCHUNK_S // 2, body, 0)
    dli, dlr = issue_loads(NCHUNK_S - 1, 0)
    dli.wait()
    dlr.wait()
    pltpu.sync_copy(rows[0], acc.at[ri[0]], add=True)
    plsc.subcore_barrier()

    @pl.when(cid == 0)
    def _():
        pltpu.sync_copy(acc.at[pl.ds(tile_row, ROWS_PER_TILE)],
                        p0_hbm.at[pl.ds(tile_row, ROWS_PER_TILE)])

    @pl.when(cid == 1)
    def _():
        pltpu.sync_copy(acc.at[pl.ds(tile_row, ROWS_PER_TILE)],
                        p1_hbm.at[pl.ds(tile_row, ROWS_PER_TILE)])

    @pl.when((sid == NS - 1) & (cid == 0))
    def _():
        pltpu.sync_copy(acc.at[pl.ds(ROWS_PER_TILE * NS, ROWS_REM)],
                        p0_hbm.at[pl.ds(ROWS_PER_TILE * NS, ROWS_REM)])

    @pl.when((sid == NS - 1) & (cid == 1))
    def _():
        pltpu.sync_copy(acc.at[pl.ds(ROWS_PER_TILE * NS, ROWS_REM)],
                        p1_hbm.at[pl.ds(ROWS_PER_TILE * NS, ROWS_REM)])


def _prep_mlp(p, pad_in=None):
    ws = [w for w in p['W']]
    bs = [b.reshape(1, -1) for b in p['b']]
    if pad_in is not None:
        ws[0] = jnp.pad(ws[0], ((0, pad_in - ws[0].shape[0]), (0, 0)))
    out = []
    for w, b in zip(ws, bs):
        out += [w, b]
    if 'ln_scale' in p:
        out += [p['ln_scale'].reshape(1, -1), p['ln_bias'].reshape(1, -1)]
    return out


def _edge_tables(blk):
    w1 = blk['edge']['W'][0]
    return w1[:D], blk['edge']['b'][0].reshape(1, -1), w1[D:2 * D]


def kernel(node_features, edge_index, edge_features, params):
    senders = edge_index[0].astype(jnp.int32)
    receivers = edge_index[1].astype(jnp.int32)
    blocks = params['blocks']

    enc_n = _prep_mlp(params['enc_node'])
    enc_e = _prep_mlp(params['enc_edge'], pad_in=8)
    ef8 = jnp.pad(edge_features, ((0, 0), (0, 4)))

    ws0, bs0, wr0 = _edge_tables(blocks[0])
    nl, ta, tb = _run_rows(_enc_node_body, [node_features],
                           enc_n + [ws0, bs0, wr0], D, 1000, n_out=3)
    el = _run_rows(_enc_edge_body, [ef8], enc_e, D, 2000)

    zeros_tab = jnp.zeros((N_NODES, D), jnp.float32)

    for s, blk in enumerate(blocks):
        pe = blk['edge']
        w1 = pe['W'][0]
        ew = [w1[2 * D:],
              pe['W'][1], pe['b'][1].reshape(1, -1),
              pe['W'][2], pe['b'][2].reshape(1, -1),
              pe['ln_scale'].reshape(1, -1), pe['ln_bias'].reshape(1, -1)]
        pn = blk['node']
        nw1 = pn['W'][0]
        nwl = [nw1[:D], nw1[D:], pn['b'][0].reshape(1, -1),
               pn['W'][1], pn['b'][1].reshape(1, -1),
               pn['W'][2], pn['b'][2].reshape(1, -1),
               pn['ln_scale'].reshape(1, -1), pn['ln_bias'].reshape(1, -1)]

        u1 = _sc_gather(ta, tb, senders, receivers)
        el = _run_rows(_edge_step_body, [u1, el], ew, D, 2000)
        p0, p1 = _sc_scatter(el, receivers, zeros_tab)
        if s + 1 < len(blocks):
            wsn, bsn, wrn = _edge_tables(blocks[s + 1])
            nl, ta, tb = _run_rows(_node_step_body, [nl, p0, p1],
                                   nwl + [wsn, bsn, wrn], D, 1000, n_out=3)
        else:
            nl = _run_rows(_node_last_body, [nl, p0, p1], nwl, D, 1000)

    dec = _prep_mlp(params['dec'])
    dec[4] = jnp.pad(dec[4], ((0, 0), (0, 5)))
    dec[5] = jnp.pad(dec[5], ((0, 0), (0, 5)))
    out8 = _run_rows(_dec_body, [nl], dec, 8, 1000)
    return out8[:, :3]


# speedup vs baseline: 2.3665x; 1.0417x over previous
import functools

import jax
import jax.numpy as jnp
from jax import lax
from jax.experimental import pallas as pl
from jax.experimental.pallas import tpu as pltpu
from jax.experimental.pallas import tpu_sc as plsc

N_NODES = 10000
N_EDGES = 320000
D = 128

NC = 2
NS = 16
NW = NC * NS
EPW = N_EDGES // NW
CHUNK = 200
NCHUNK = EPW // CHUNK
EH = N_EDGES // 2
EPW_S = EH // NW
CHUNK_S = 40
NCHUNK_S = EPW_S // CHUNK_S
ROWS_PER_TILE = 624
ROWS_REM = N_NODES - ROWS_PER_TILE * NS


def _ln(x, g, b):
    mu = jnp.mean(x, axis=-1, keepdims=True)
    var = jnp.mean((x - mu) ** 2, axis=-1, keepdims=True)
    return (x - mu) / jnp.sqrt(var + 1e-5) * g + b


def _dot(x, w):
    return jnp.dot(x, w, preferred_element_type=jnp.float32,
                   precision=jax.lax.Precision.HIGHEST)


def _enc_node_body(x_ref, w0, b0, w1, b1, w2, b2, g, be, ws, bs, wr,
                   o_ref, a_ref, t_ref):
    h = jax.nn.relu(_dot(x_ref[...], w0[...]) + b0[...])
    h = jax.nn.relu(_dot(h, w1[...]) + b1[...])
    h = _dot(h, w2[...]) + b2[...]
    nl = _ln(h, g[...], be[...])
    o_ref[...] = nl
    a_ref[...] = _dot(nl, ws[...]) + bs[...]
    t_ref[...] = _dot(nl, wr[...])


def _enc_edge_body(x_ref, w0, b0, w1, b1, w2, b2, g, be, o_ref):
    h = jax.nn.relu(_dot(x_ref[...], w0[...]) + b0[...])
    h = jax.nn.relu(_dot(h, w1[...]) + b1[...])
    h = _dot(h, w2[...]) + b2[...]
    o_ref[...] = _ln(h, g[...], be[...])


def _edge_step_body(u1_ref, el_ref, we, w1, bb1, w2, b2, g, be, o_ref):
    el = el_ref[...]
    h = jax.nn.relu(u1_ref[...] + _dot(el, we[...]))
    h = jax.nn.relu(_dot(h, w1[...]) + bb1[...])
    h = _dot(h, w2[...]) + b2[...]
    o_ref[...] = el + _ln(h, g[...], be[...])


def _node_step_body(nl_ref, p0_ref, p1_ref, p2_ref, p3_ref,
                    wa, wb, b0, w1, b1, w2, b2,
                    g, be, ws, bs, wr, o_ref, a_ref, t_ref):
    nl = nl_ref[...]
    agg = (p0_ref[...] + p1_ref[...]) + (p2_ref[...] + p3_ref[...])
    h = jax.nn.relu(_dot(nl, wa[...]) + _dot(agg, wb[...]) + b0[...])
    h = jax.nn.relu(_dot(h, w1[...]) + b1[...])
    h = _dot(h, w2[...]) + b2[...]
    nl = nl + _ln(h, g[...], be[...])
    o_ref[...] = nl
    a_ref[...] = _dot(nl, ws[...]) + bs[...]
    t_ref[...] = _dot(nl, wr[...])


def _node_last_body(nl_ref, p0_ref, p1_ref, p2_ref, p3_ref,
                    wa, wb, b0, w1, b1, w2, b2, g, be, o_ref):
    nl = nl_ref[...]
    agg = (p0_ref[...] + p1_ref[...]) + (p2_ref[...] + p3_ref[...])
    h = jax.nn.relu(_dot(nl, wa[...]) + _dot(agg, wb[...]) + b0[...])
    h = jax.nn.relu(_dot(h, w1[...]) + b1[...])
    h = _dot(h, w2[...]) + b2[...]
    o_ref[...] = nl + _ln(h, g[...], be[...])


def _dec_body(x_ref, w0, b0, w1, b1, w2, b2, o_ref):
    h = jax.nn.relu(_dot(x_ref[...], w0[...]) + b0[...])
    h = jax.nn.relu(_dot(h, w1[...]) + b1[...])
    o_ref[...] = _dot(h, w2[...]) + b2[...]


def _full_spec(shape):
    nd = len(shape)
    return pl.BlockSpec(shape, lambda i: (0,) * nd)


def _row_spec(bm, cols):
    return pl.BlockSpec((bm, cols), lambda i: (i, 0))


def _off_spec(bm, cols, off):
    return pl.BlockSpec((bm, cols), lambda i, _o=off: (i + _o, 0))


def _run_rows(body, x_list, w_list, out_cols, bm, n_out=1,
              x_offs=None, out_rows=None):
    rows = out_rows if out_rows is not None else x_list[0].shape[0]
    grid = rows // bm
    if x_offs is None:
        x_offs = [0] * len(x_list)
    in_specs = [_off_spec(bm, x.shape[1], o) for x, o in zip(x_list, x_offs)] \
        + [_full_spec(w.shape) for w in w_list]
    if n_out == 1:
        out_specs = _row_spec(bm, out_cols)
        out_shape = jax.ShapeDtypeStruct((rows, out_cols), jnp.float32)
    else:
        out_specs = [_row_spec(bm, out_cols) for _ in range(n_out)]
        out_shape = [jax.ShapeDtypeStruct((rows, out_cols), jnp.float32)
                     for _ in range(n_out)]
    return pl.pallas_call(
        body,
        grid=(grid,),
        in_specs=in_specs,
        out_specs=out_specs,
        out_shape=out_shape,
    )(*x_list, *w_list)


_MESH = plsc.VectorSubcoreMesh(core_axis_name="c", subcore_axis_name="s")


@functools.partial(
    pl.kernel,
    out_type=jax.ShapeDtypeStruct((N_EDGES, D), jnp.float32),
    mesh=_MESH,
    scratch_types=[
        [pltpu.VMEM((CHUNK,), jnp.int32) for _ in range(2)],
        [pltpu.VMEM((CHUNK,), jnp.int32) for _ in range(2)],
        [pltpu.VMEM((CHUNK, D), jnp.float32) for _ in range(2)],
        [pltpu.VMEM((CHUNK, D), jnp.float32) for _ in range(2)],
        [pltpu.SemaphoreType.DMA for _ in range(2)],
        [pltpu.SemaphoreType.DMA for _ in range(2)],
        [pltpu.SemaphoreType.DMA for _ in range(2)],
    ],
)
def _sc_gather(a_hbm, b_hbm, s_hbm, r_hbm, u1_hbm,
               si, ri, ra, rb, semi, semg, sems):
    wid = lax.axis_index("s") * NC + lax.axis_index("c")
    base = wid * EPW

    def issue_idx(c, t):
        off = pl.multiple_of(base + c * CHUNK, 8)
        pltpu.async_copy(s_hbm.at[pl.ds(off, CHUNK)], si[t], semi[t])
        pltpu.async_copy(r_hbm.at[pl.ds(off, CHUNK)], ri[t], semi[t])

    def wait_idx(t):
        pltpu.make_async_copy(s_hbm.at[pl.ds(0, CHUNK)], si[t], semi[t]).wait()
        pltpu.make_async_copy(r_hbm.at[pl.ds(0, CHUNK)], ri[t], semi[t]).wait()

    def issue_gather(t):
        pltpu.async_copy(a_hbm.at[si[t]], ra[t], semg[t])
        pltpu.async_copy(b_hbm.at[ri[t]], rb[t], semg[t])

    def wait_gather(t):
        pltpu.make_async_copy(u1_hbm.at[pl.ds(0, CHUNK)], ra[t], semg[t]).wait()
        pltpu.make_async_copy(u1_hbm.at[pl.ds(0, CHUNK)], rb[t], semg[t]).wait()

    def issue_store(c, t):
        off = pl.multiple_of(base + c * CHUNK, 8)
        pltpu.async_copy(ra[t], u1_hbm.at[pl.ds(off, CHUNK)], sems[t])

    def wait_store(t):
        pltpu.make_async_copy(ra[t], u1_hbm.at[pl.ds(0, CHUNK)], sems[t]).wait()

    def add(t):
        def add_row(e, c):
            for k in range(D // 16):
                sl = pl.ds(k * 16, 16)
                ra[t][e, sl] = ra[t][e, sl] + rb[t][e, sl]
            return c
        lax.fori_loop(0, CHUNK, add_row, 0)

    issue_idx(0, 0)
    issue_idx(1, 1)
    wait_idx(0)
    issue_gather(0)
    wait_idx(1)
    issue_gather(1)

    def body(k, carry):
        c = 2 * k
        wait_gather(0)
        issue_idx(c + 2, 0)
        add(0)
        issue_store(c, 0)
        wait_gather(1)
        issue_idx(c + 3, 1)
        wait_store(0)
        wait_idx(0)
        issue_gather(0)
        add(1)
        issue_store(c + 1, 1)
        wait_store(1)
        wait_idx(1)
        issue_gather(1)
        return carry

    lax.fori_loop(0, NCHUNK // 2 - 1, body, 0)

    wait_gather(0)
    add(0)
    issue_store(NCHUNK - 2, 0)
    wait_gather(1)
    add(1)
    issue_store(NCHUNK - 1, 1)
    wait_store(0)
    wait_store(1)


@functools.partial(
    pl.kernel,
    out_type=[jax.ShapeDtypeStruct((N_NODES, D), jnp.float32),
              jax.ShapeDtypeStruct((N_NODES, D), jnp.float32)],
    mesh=_MESH,
    scratch_types=[
        [pltpu.VMEM((CHUNK_S,), jnp.int32) for _ in range(2)],
        [pltpu.VMEM((CHUNK_S, D), jnp.float32) for _ in range(2)],
        pltpu.VMEM_SHARED((N_NODES, D), jnp.float32),
        [pltpu.SemaphoreType.DMA for _ in range(2)],
    ],
)
def _sc_scatter(ne_hbm, r_hbm, z_hbm, p0_hbm, p1_hbm, ri, rows, acc, seml):
    cid = lax.axis_index("c")
    sid = lax.axis_index("s")
    wid = sid * NC + cid
    base = wid * EPW_S
    tile_row = sid * ROWS_PER_TILE

    pltpu.sync_copy(z_hbm.at[pl.ds(tile_row, ROWS_PER_TILE)],
                    acc.at[pl.ds(tile_row, ROWS_PER_TILE)])

    @pl.when(sid == NS - 1)
    def _():
        pltpu.sync_copy(z_hbm.at[pl.ds(ROWS_PER_TILE * NS, ROWS_REM)],
                        acc.at[pl.ds(ROWS_PER_TILE * NS, ROWS_REM)])

    plsc.subcore_barrier()

    def issue_loads(c, t):
        off = pl.multiple_of(base + c * CHUNK_S, 8)
        pltpu.async_copy(r_hbm.at[pl.ds(off, CHUNK_S)], ri[t], seml[t])
        pltpu.async_copy(ne_hbm.at[pl.ds(off, CHUNK_S)], rows[t], seml[t])

    def wait_loads(t):
        pltpu.make_async_copy(r_hbm.at[pl.ds(0, CHUNK_S)], ri[t],
                              seml[t]).wait()
        pltpu.make_async_copy(ne_hbm.at[pl.ds(0, CHUNK_S)], rows[t],
                              seml[t]).wait()

    def issue_loads(c, t):
        off = pl.multiple_of(base + c * CHUNK_S, 8)
        di = pltpu.async_copy(r_hbm.at[pl.ds(off, CHUNK_S)], ri[t], seml[t])
        dr = pltpu.async_copy(ne_hbm.at[pl.ds(off, CHUNK_S)], rows[t],
                              seml[t])
        return di, dr

    def body(k, carry):
        c = 2 * k
        d0i, d0r = issue_loads(c, 0)
        d1i, d1r = issue_loads(c + 1, 1)
        d0i.wait()
        d0r.wait()
        pltpu.sync_copy(rows[0], acc.at[ri[0]], add=True)
        d1i.wait()
        d1r.wait()
        pltpu.sync_copy(rows[1], acc.at[ri[1]], add=True)
        return carry

    lax.fori_loop(0, NCHUNK_S // 2, body, 0)
    dli, dlr = issue_loads(NCHUNK_S - 1, 0)
    dli.wait()
    dlr.wait()
    pltpu.sync_copy(rows[0], acc.at[ri[0]], add=True)
    plsc.subcore_barrier()

    @pl.when(cid == 0)
    def _():
        pltpu.sync_copy(acc.at[pl.ds(tile_row, ROWS_PER_TILE)],
                        p0_hbm.at[pl.ds(tile_row, ROWS_PER_TILE)])

    @pl.when(cid == 1)
    def _():
        pltpu.sync_copy(acc.at[pl.ds(tile_row, ROWS_PER_TILE)],
                        p1_hbm.at[pl.ds(tile_row, ROWS_PER_TILE)])

    @pl.when((sid == NS - 1) & (cid == 0))
    def _():
        pltpu.sync_copy(acc.at[pl.ds(ROWS_PER_TILE * NS, ROWS_REM)],
                        p0_hbm.at[pl.ds(ROWS_PER_TILE * NS, ROWS_REM)])

    @pl.when((sid == NS - 1) & (cid == 1))
    def _():
        pltpu.sync_copy(acc.at[pl.ds(ROWS_PER_TILE * NS, ROWS_REM)],
                        p1_hbm.at[pl.ds(ROWS_PER_TILE * NS, ROWS_REM)])


def _prep_mlp(p, pad_in=None):
    ws = [w for w in p['W']]
    bs = [b.reshape(1, -1) for b in p['b']]
    if pad_in is not None:
        ws[0] = jnp.pad(ws[0], ((0, pad_in - ws[0].shape[0]), (0, 0)))
    out = []
    for w, b in zip(ws, bs):
        out += [w, b]
    if 'ln_scale' in p:
        out += [p['ln_scale'].reshape(1, -1), p['ln_bias'].reshape(1, -1)]
    return out


def _edge_tables(blk):
    w1 = blk['edge']['W'][0]
    return w1[:D], blk['edge']['b'][0].reshape(1, -1), w1[D:2 * D]


def kernel(node_features, edge_index, edge_features, params):
    senders = edge_index[0].astype(jnp.int32)
    receivers = edge_index[1].astype(jnp.int32)
    blocks = params['blocks']

    enc_n = _prep_mlp(params['enc_node'])
    enc_e = _prep_mlp(params['enc_edge'], pad_in=8)
    ef8 = jnp.pad(edge_features, ((0, 0), (0, 4)))

    ws0, bs0, wr0 = _edge_tables(blocks[0])
    nl, ta, tb = _run_rows(_enc_node_body, [node_features],
                           enc_n + [ws0, bs0, wr0], D, 1000, n_out=3)
    BE = 2000
    HB = EH // BE
    el0 = _run_rows(_enc_edge_body, [ef8], enc_e, D, BE, out_rows=EH)
    el1 = _run_rows(_enc_edge_body, [ef8], enc_e, D, BE, out_rows=EH,
                    x_offs=[HB])

    r0 = receivers[:EH]
    r1 = receivers[EH:]
    zeros_tab = jnp.zeros((N_NODES, D), jnp.float32)

    for s, blk in enumerate(blocks):
        pe = blk['edge']
        w1 = pe['W'][0]
        ew = [w1[2 * D:],
              pe['W'][1], pe['b'][1].reshape(1, -1),
              pe['W'][2], pe['b'][2].reshape(1, -1),
              pe['ln_scale'].reshape(1, -1), pe['ln_bias'].reshape(1, -1)]
        pn = blk['node']
        nw1 = pn['W'][0]
        nwl = [nw1[:D], nw1[D:], pn['b'][0].reshape(1, -1),
               pn['W'][1], pn['b'][1].reshape(1, -1),
               pn['W'][2], pn['b'][2].reshape(1, -1),
               pn['ln_scale'].reshape(1, -1), pn['ln_bias'].reshape(1, -1)]

        u1 = _sc_gather(ta, tb, senders, receivers)
        el0 = _run_rows(_edge_step_body, [u1, el0], ew, D, BE, out_rows=EH)
        p0, p1 = _sc_scatter(el0, r0, zeros_tab)
        el1 = _run_rows(_edge_step_body, [u1, el1], ew, D, BE, out_rows=EH,
                        x_offs=[HB, 0])
        p2, p3 = _sc_scatter(el1, r1, zeros_tab)
        if s + 1 < len(blocks):
            wsn, bsn, wrn = _edge_tables(blocks[s + 1])
            nl, ta, tb = _run_rows(_node_step_body, [nl, p0, p1, p2, p3],
                                   nwl + [wsn, bsn, wrn], D, 1000, n_out=3)
        else:
            nl = _run_rows(_node_last_body, [nl, p0, p1, p2, p3],
                           nwl, D, 1000)

    dec = _prep_mlp(params['dec'])
    dec[4] = jnp.pad(dec[4], ((0, 0), (0, 5)))
    dec[5] = jnp.pad(dec[5], ((0, 0), (0, 5)))
    out8 = _run_rows(_dec_body, [nl], dec, 8, 1000)
    return out8[:, :3]


# speedup vs baseline: 2.4701x; 1.0438x over previous
import functools

import jax
import jax.numpy as jnp
from jax import lax
from jax.experimental import pallas as pl
from jax.experimental.pallas import tpu as pltpu
from jax.experimental.pallas import tpu_sc as plsc

N_NODES = 10000
N_EDGES = 320000
D = 128

NC = 2
NS = 16
NW = NC * NS
EPW = N_EDGES // NW
CHUNK = 200
NCHUNK = EPW // CHUNK
EH = N_EDGES // 2
EPW_S = EH // NW
CHUNK_S = 40
NCHUNK_S = EPW_S // CHUNK_S
ROWS_PER_TILE = 624
ROWS_REM = N_NODES - ROWS_PER_TILE * NS


def _split(x):
    xh = x.astype(jnp.bfloat16)
    xl = (x - xh.astype(jnp.float32)).astype(jnp.bfloat16)
    return xh, xl


def _dot(x, wp):
    xh, xl = _split(x)
    wh = wp[0]
    wl = wp[1]
    return (jnp.dot(xh, wh, preferred_element_type=jnp.float32)
            + (jnp.dot(xh, wl, preferred_element_type=jnp.float32)
               + jnp.dot(xl, wh, preferred_element_type=jnp.float32)))


def _ln(x, g, b, mm):
    xh, xl = _split(x)
    mu = (jnp.dot(xh, mm, preferred_element_type=jnp.float32)
          + jnp.dot(xl, mm, preferred_element_type=jnp.float32))
    xc = x - mu
    s = xc * xc
    sh, sl = _split(s)
    var = (jnp.dot(sh, mm, preferred_element_type=jnp.float32)
           + jnp.dot(sl, mm, preferred_element_type=jnp.float32))
    return xc / jnp.sqrt(var + 1e-5) * g + b


def _enc_node_body(x_ref, mm, w0, b0, w1, b1, w2, b2, g, be, ws, bs, wr,
                   o_ref, a_ref, t_ref):
    h = jax.nn.relu(_dot(x_ref[...], w0[...]) + b0[...])
    h = jax.nn.relu(_dot(h, w1[...]) + b1[...])
    h = _dot(h, w2[...]) + b2[...]
    nl = _ln(h, g[...], be[...], mm[...])
    o_ref[...] = nl
    a_ref[...] = _dot(nl, ws[...]) + bs[...]
    t_ref[...] = _dot(nl, wr[...])


def _enc_edge_body(x_ref, mm, w0, b0, w1, b1, w2, b2, g, be, o_ref):
    h = jax.nn.relu(_dot(x_ref[...], w0[...]) + b0[...])
    h = jax.nn.relu(_dot(h, w1[...]) + b1[...])
    h = _dot(h, w2[...]) + b2[...]
    o_ref[...] = _ln(h, g[...], be[...], mm[...])


def _edge_step_body(u1_ref, el_ref, mm, we, w1, bb1, w2, b2, g, be, o_ref):
    el = el_ref[...]
    h = jax.nn.relu(u1_ref[...] + _dot(el, we[...]))
    h = jax.nn.relu(_dot(h, w1[...]) + bb1[...])
    h = _dot(h, w2[...]) + b2[...]
    o_ref[...] = el + _ln(h, g[...], be[...], mm[...])


def _node_step_body(nl_ref, p0_ref, p1_ref, p2_ref, p3_ref, mm,
                    wa, wb, b0, w1, b1, w2, b2,
                    g, be, ws, bs, wr, o_ref, a_ref, t_ref):
    nl = nl_ref[...]
    agg = (p0_ref[...] + p1_ref[...]) + (p2_ref[...] + p3_ref[...])
    h = jax.nn.relu(_dot(nl, wa[...]) + _dot(agg, wb[...]) + b0[...])
    h = jax.nn.relu(_dot(h, w1[...]) + b1[...])
    h = _dot(h, w2[...]) + b2[...]
    nl = nl + _ln(h, g[...], be[...], mm[...])
    o_ref[...] = nl
    a_ref[...] = _dot(nl, ws[...]) + bs[...]
    t_ref[...] = _dot(nl, wr[...])


def _node_last_body(nl_ref, p0_ref, p1_ref, p2_ref, p3_ref, mm,
                    wa, wb, b0, w1, b1, w2, b2, g, be, o_ref):
    nl = nl_ref[...]
    agg = (p0_ref[...] + p1_ref[...]) + (p2_ref[...] + p3_ref[...])
    h = jax.nn.relu(_dot(nl, wa[...]) + _dot(agg, wb[...]) + b0[...])
    h = jax.nn.relu(_dot(h, w1[...]) + b1[...])
    h = _dot(h, w2[...]) + b2[...]
    o_ref[...] = nl + _ln(h, g[...], be[...], mm[...])


def _dec_body(x_ref, mm, w0, b0, w1, b1, w2, b2, o_ref):
    h = jax.nn.relu(_dot(x_ref[...], w0[...]) + b0[...])
    h = jax.nn.relu(_dot(h, w1[...]) + b1[...])
    o_ref[...] = _dot(h, w2[...]) + b2[...]


def _full_spec(shape):
    nd = len(shape)
    return pl.BlockSpec(shape, lambda i: (0,) * nd)


def _row_spec(bm, cols):
    return pl.BlockSpec((bm, cols), lambda i: (i, 0))


def _off_spec(bm, cols, off):
    return pl.BlockSpec((bm, cols), lambda i, _o=off: (i + _o, 0))


def _run_rows(body, x_list, w_list, out_cols, bm, n_out=1,
              x_offs=None, out_rows=None):
    rows = out_rows if out_rows is not None else x_list[0].shape[0]
    grid = rows // bm
    if x_offs is None:
        x_offs = [0] * len(x_list)
    in_specs = [_off_spec(bm, x.shape[1], o) for x, o in zip(x_list, x_offs)] \
        + [_full_spec(w.shape) for w in w_list]
    if n_out == 1:
        out_specs = _row_spec(bm, out_cols)
        out_shape = jax.ShapeDtypeStruct((rows, out_cols), jnp.float32)
    else:
        out_specs = [_row_spec(bm, out_cols) for _ in range(n_out)]
        out_shape = [jax.ShapeDtypeStruct((rows, out_cols), jnp.float32)
                     for _ in range(n_out)]
    return pl.pallas_call(
        body,
        grid=(grid,),
        in_specs=in_specs,
        out_specs=out_specs,
        out_shape=out_shape,
    )(*x_list, *w_list)


_MESH = plsc.VectorSubcoreMesh(core_axis_name="c", subcore_axis_name="s")


@functools.partial(
    pl.kernel,
    out_type=jax.ShapeDtypeStruct((N_EDGES, D), jnp.float32),
    mesh=_MESH,
    scratch_types=[
        [pltpu.VMEM((CHUNK,), jnp.int32) for _ in range(2)],
        [pltpu.VMEM((CHUNK,), jnp.int32) for _ in range(2)],
        [pltpu.VMEM((CHUNK, D), jnp.float32) for _ in range(2)],
        [pltpu.VMEM((CHUNK, D), jnp.float32) for _ in range(2)],
        [pltpu.SemaphoreType.DMA for _ in range(2)],
        [pltpu.SemaphoreType.DMA for _ in range(2)],
        [pltpu.SemaphoreType.DMA for _ in range(2)],
    ],
)
def _sc_gather(a_hbm, b_hbm, s_hbm, r_hbm, u1_hbm,
               si, ri, ra, rb, semi, semg, sems):
    wid = lax.axis_index("s") * NC + lax.axis_index("c")
    base = wid * EPW

    def issue_idx(c, t):
        off = pl.multiple_of(base + c * CHUNK, 8)
        pltpu.async_copy(s_hbm.at[pl.ds(off, CHUNK)], si[t], semi[t])
        pltpu.async_copy(r_hbm.at[pl.ds(off, CHUNK)], ri[t], semi[t])

    def wait_idx(t):
        pltpu.make_async_copy(s_hbm.at[pl.ds(0, CHUNK)], si[t], semi[t]).wait()
        pltpu.make_async_copy(r_hbm.at[pl.ds(0, CHUNK)], ri[t], semi[t]).wait()

    def issue_gather(t):
        pltpu.async_copy(a_hbm.at[si[t]], ra[t], semg[t])
        pltpu.async_copy(b_hbm.at[ri[t]], rb[t], semg[t])

    def wait_gather(t):
        pltpu.make_async_copy(u1_hbm.at[pl.ds(0, CHUNK)], ra[t], semg[t]).wait()
        pltpu.make_async_copy(u1_hbm.at[pl.ds(0, CHUNK)], rb[t], semg[t]).wait()

    def issue_store(c, t):
        off = pl.multiple_of(base + c * CHUNK, 8)
        pltpu.async_copy(ra[t], u1_hbm.at[pl.ds(off, CHUNK)], sems[t])

    def wait_store(t):
        pltpu.make_async_copy(ra[t], u1_hbm.at[pl.ds(0, CHUNK)], sems[t]).wait()

    def add(t):
        def add_row(e, c):
            for k in range(D // 16):
                sl = pl.ds(k * 16, 16)
                ra[t][e, sl] = ra[t][e, sl] + rb[t][e, sl]
            return c
        lax.fori_loop(0, CHUNK, add_row, 0)

    issue_idx(0, 0)
    issue_idx(1, 1)
    wait_idx(0)
    issue_gather(0)
    wait_idx(1)
    issue_gather(1)

    def body(k, carry):
        c = 2 * k
        wait_gather(0)
        issue_idx(c + 2, 0)
        add(0)
        issue_store(c, 0)
        wait_gather(1)
        issue_idx(c + 3, 1)
        wait_store(0)
        wait_idx(0)
        issue_gather(0)
        add(1)
        issue_store(c + 1, 1)
        wait_store(1)
        wait_idx(1)
        issue_gather(1)
        return carry

    lax.fori_loop(0, NCHUNK // 2 - 1, body, 0)

    wait_gather(0)
    add(0)
    issue_store(NCHUNK - 2, 0)
    wait_gather(1)
    add(1)
    issue_store(NCHUNK - 1, 1)
    wait_store(0)
    wait_store(1)


@functools.partial(
    pl.kernel,
    out_type=[jax.ShapeDtypeStruct((N_NODES, D), jnp.float32),
              jax.ShapeDtypeStruct((N_NODES, D), jnp.float32)],
    mesh=_MESH,
    scratch_types=[
        [pltpu.VMEM((CHUNK_S,), jnp.int32) for _ in range(2)],
        [pltpu.VMEM((CHUNK_S, D), jnp.float32) for _ in range(2)],
        pltpu.VMEM_SHARED((N_NODES, D), jnp.float32),
        [pltpu.SemaphoreType.DMA for _ in range(2)],
    ],
)
def _sc_scatter(ne_hbm, r_hbm, z_hbm, p0_hbm, p1_hbm, ri, rows, acc, seml):
    cid = lax.axis_index("c")
    sid = lax.axis_index("s")
    wid = sid * NC + cid
    base = wid * EPW_S
    tile_row = sid * ROWS_PER_TILE

    pltpu.sync_copy(z_hbm.at[pl.ds(tile_row, ROWS_PER_TILE)],
                    acc.at[pl.ds(tile_row, ROWS_PER_TILE)])

    @pl.when(sid == NS - 1)
    def _():
        pltpu.sync_copy(z_hbm.at[pl.ds(ROWS_PER_TILE * NS, ROWS_REM)],
                        acc.at[pl.ds(ROWS_PER_TILE * NS, ROWS_REM)])

    plsc.subcore_barrier()

    def issue_loads(c, t):
        off = pl.multiple_of(base + c * CHUNK_S, 8)
        pltpu.async_copy(r_hbm.at[pl.ds(off, CHUNK_S)], ri[t], seml[t])
        pltpu.async_copy(ne_hbm.at[pl.ds(off, CHUNK_S)], rows[t], seml[t])

    def wait_loads(t):
        pltpu.make_async_copy(r_hbm.at[pl.ds(0, CHUNK_S)], ri[t],
                              seml[t]).wait()
        pltpu.make_async_copy(ne_hbm.at[pl.ds(0, CHUNK_S)], rows[t],
                              seml[t]).wait()

    def issue_loads(c, t):
        off = pl.multiple_of(base + c * CHUNK_S, 8)
        di = pltpu.async_copy(r_hbm.at[pl.ds(off, CHUNK_S)], ri[t], seml[t])
        dr = pltpu.async_copy(ne_hbm.at[pl.ds(off, CHUNK_S)], rows[t],
                              seml[t])
        return di, dr

    def body(k, carry):
        c = 2 * k
        d0i, d0r = issue_loads(c, 0)
        d1i, d1r = issue_loads(c + 1, 1)
        d0i.wait()
        d0r.wait()
        pltpu.sync_copy(rows[0], acc.at[ri[0]], add=True)
        d1i.wait()
        d1r.wait()
        pltpu.sync_copy(rows[1], acc.at[ri[1]], add=True)
        return carry

    lax.fori_loop(0, NCHUNK_S // 2, body, 0)
    dli, dlr = issue_loads(NCHUNK_S - 1, 0)
    dli.wait()
    dlr.wait()
    pltpu.sync_copy(rows[0], acc.at[ri[0]], add=True)
    plsc.subcore_barrier()

    @pl.when(cid == 0)
    def _():
        pltpu.sync_copy(acc.at[pl.ds(tile_row, ROWS_PER_TILE)],
                        p0_hbm.at[pl.ds(tile_row, ROWS_PER_TILE)])

    @pl.when(cid == 1)
    def _():
        pltpu.sync_copy(acc.at[pl.ds(tile_row, ROWS_PER_TILE)],
                        p1_hbm.at[pl.ds(tile_row, ROWS_PER_TILE)])

    @pl.when((sid == NS - 1) & (cid == 0))
    def _():
        pltpu.sync_copy(acc.at[pl.ds(ROWS_PER_TILE * NS, ROWS_REM)],
                        p0_hbm.at[pl.ds(ROWS_PER_TILE * NS, ROWS_REM)])

    @pl.when((sid == NS - 1) & (cid == 1))
    def _():
        pltpu.sync_copy(acc.at[pl.ds(ROWS_PER_TILE * NS, ROWS_REM)],
                        p1_hbm.at[pl.ds(ROWS_PER_TILE * NS, ROWS_REM)])


def _pair(w):
    hi = w.astype(jnp.bfloat16)
    lo = (w - hi.astype(jnp.float32)).astype(jnp.bfloat16)
    return jnp.stack([hi, lo])


def _prep_mlp(p, pad_in=None):
    ws = [w for w in p['W']]
    bs = [b.reshape(1, -1) for b in p['b']]
    if pad_in is not None:
        ws[0] = jnp.pad(ws[0], ((0, pad_in - ws[0].shape[0]), (0, 0)))
    out = []
    for w, b in zip(ws, bs):
        out += [_pair(w), b]
    if 'ln_scale' in p:
        out += [p['ln_scale'].reshape(1, -1), p['ln_bias'].reshape(1, -1)]
    return out


def _edge_tables(blk):
    w1 = blk['edge']['W'][0]
    return (_pair(w1[:D]), blk['edge']['b'][0].reshape(1, -1),
            _pair(w1[D:2 * D]))


def kernel(node_features, edge_index, edge_features, params):
    senders = edge_index[0].astype(jnp.int32)
    receivers = edge_index[1].astype(jnp.int32)
    blocks = params['blocks']

    enc_n = _prep_mlp(params['enc_node'])
    enc_e = _prep_mlp(params['enc_edge'], pad_in=8)
    ef8 = jnp.pad(edge_features, ((0, 0), (0, 4)))

    mm = jnp.full((D, D), 1.0 / D, jnp.bfloat16)
    ws0, bs0, wr0 = _edge_tables(blocks[0])
    nl, ta, tb = _run_rows(_enc_node_body, [node_features],
                           [mm] + enc_n + [ws0, bs0, wr0], D, 1000, n_out=3)
    BE = 2000
    HB = EH // BE
    el0 = _run_rows(_enc_edge_body, [ef8], [mm] + enc_e, D, BE, out_rows=EH)
    el1 = _run_rows(_enc_edge_body, [ef8], [mm] + enc_e, D, BE, out_rows=EH,
                    x_offs=[HB])

    r0 = receivers[:EH]
    r1 = receivers[EH:]
    zeros_tab = jnp.zeros((N_NODES, D), jnp.float32)

    for s, blk in enumerate(blocks):
        pe = blk['edge']
        w1 = pe['W'][0]
        ew = [mm, _pair(w1[2 * D:]),
              _pair(pe['W'][1]), pe['b'][1].reshape(1, -1),
              _pair(pe['W'][2]), pe['b'][2].reshape(1, -1),
              pe['ln_scale'].reshape(1, -1), pe['ln_bias'].reshape(1, -1)]
        pn = blk['node']
        nw1 = pn['W'][0]
        nwl = [mm, _pair(nw1[:D]), _pair(nw1[D:]),
               pn['b'][0].reshape(1, -1),
               _pair(pn['W'][1]), pn['b'][1].reshape(1, -1),
               _pair(pn['W'][2]), pn['b'][2].reshape(1, -1),
               pn['ln_scale'].reshape(1, -1), pn['ln_bias'].reshape(1, -1)]

        u1 = _sc_gather(ta, tb, senders, receivers)
        el0 = _run_rows(_edge_step_body, [u1, el0], ew, D, BE, out_rows=EH)
        p0, p1 = _sc_scatter(el0, r0, zeros_tab)
        el1 = _run_rows(_edge_step_body, [u1, el1], ew, D, BE, out_rows=EH,
                        x_offs=[HB, 0])
        p2, p3 = _sc_scatter(el1, r1, zeros_tab)
        if s + 1 < len(blocks):
            wsn, bsn, wrn = _edge_tables(blocks[s + 1])
            nl, ta, tb = _run_rows(_node_step_body, [nl, p0, p1, p2, p3],
                                   nwl + [wsn, bsn, wrn], D, 1000, n_out=3)
        else:
            nl = _run_rows(_node_last_body, [nl, p0, p1, p2, p3],
                           nwl, D, 1000)

    pd = params['dec']
    dec = [mm,
           _pair(pd['W'][0]), pd['b'][0].reshape(1, -1),
           _pair(pd['W'][1]), pd['b'][1].reshape(1, -1),
           _pair(jnp.pad(pd['W'][2], ((0, 0), (0, 5)))),
           jnp.pad(pd['b'][2].reshape(1, -1), ((0, 0), (0, 5)))]
    out8 = _run_rows(_dec_body, [nl], dec, 8, 1000)
    return out8[:, :3]


# speedup vs baseline: 3.4510x; 1.3971x over previous
import functools

import jax
import jax.numpy as jnp
from jax import lax
from jax.experimental import pallas as pl
from jax.experimental.pallas import tpu as pltpu
from jax.experimental.pallas import tpu_sc as plsc

N_NODES = 10000
N_EDGES = 320000
D = 128

NC = 2
NS = 16
NW = NC * NS
EPW = N_EDGES // NW
CHUNK = 200
NCHUNK = EPW // CHUNK
EH = N_EDGES // 2
EPW_S = EH // NW
CHUNK_S = 40
NCHUNK_S = EPW_S // CHUNK_S
ROWS_PER_TILE = 624
ROWS_REM = N_NODES - ROWS_PER_TILE * NS


def _split(x):
    xh = x.astype(jnp.bfloat16)
    xl = (x - xh.astype(jnp.float32)).astype(jnp.bfloat16)
    return xh, xl


def _wide(x):
    xh, xl = _split(x)
    return jnp.concatenate([xh, xl], axis=-1)


def _dot(x, wp):
    xc = _wide(x)
    return (jnp.dot(xc, wp[0], preferred_element_type=jnp.float32)
            + jnp.dot(xc, wp[1], preferred_element_type=jnp.float32))


def _ln(x, g, b, mm):
    mu = jnp.dot(_wide(x), mm, preferred_element_type=jnp.float32)
    xc = x - mu
    var = jnp.dot(_wide(xc * xc), mm, preferred_element_type=jnp.float32)
    return xc / jnp.sqrt(var + 1e-5) * g + b


def _enc_node_body(x_ref, mm, w0, b0, w1, b1, w2, b2, g, be, ws, bs, wr,
                   o_ref, a_ref, t_ref):
    h = jax.nn.relu(_dot(x_ref[...], w0[...]) + b0[...])
    h = jax.nn.relu(_dot(h, w1[...]) + b1[...])
    h = _dot(h, w2[...]) + b2[...]
    nl = _ln(h, g[...], be[...], mm[...])
    o_ref[...] = nl
    a_ref[...] = _dot(nl, ws[...]) + bs[...]
    t_ref[...] = _dot(nl, wr[...])


def _enc_edge_body(x_ref, mm, w0, b0, w1, b1, w2, b2, g, be, o_ref):
    h = jax.nn.relu(_dot(x_ref[...], w0[...]) + b0[...])
    h = jax.nn.relu(_dot(h, w1[...]) + b1[...])
    h = _dot(h, w2[...]) + b2[...]
    o_ref[...] = _ln(h, g[...], be[...], mm[...])


def _edge_step_body(u1_ref, el_ref, mm, we, w1, bb1, w2, b2, g, be, o_ref):
    el = el_ref[...]
    h = jax.nn.relu(u1_ref[...] + _dot(el, we[...]))
    h = jax.nn.relu(_dot(h, w1[...]) + bb1[...])
    h = _dot(h, w2[...]) + b2[...]
    o_ref[...] = el + _ln(h, g[...], be[...], mm[...])


def _node_step_body(nl_ref, p0_ref, p1_ref, p2_ref, p3_ref, mm,
                    wa, wb, b0, w1, b1, w2, b2,
                    g, be, ws, bs, wr, o_ref, a_ref, t_ref):
    nl = nl_ref[...]
    agg = (p0_ref[...] + p1_ref[...]) + (p2_ref[...] + p3_ref[...])
    h = jax.nn.relu(_dot(nl, wa[...]) + _dot(agg, wb[...]) + b0[...])
    h = jax.nn.relu(_dot(h, w1[...]) + b1[...])
    h = _dot(h, w2[...]) + b2[...]
    nl = nl + _ln(h, g[...], be[...], mm[...])
    o_ref[...] = nl
    a_ref[...] = _dot(nl, ws[...]) + bs[...]
    t_ref[...] = _dot(nl, wr[...])


def _node_last_body(nl_ref, p0_ref, p1_ref, p2_ref, p3_ref, mm,
                    wa, wb, b0, w1, b1, w2, b2, g, be, o_ref):
    nl = nl_ref[...]
    agg = (p0_ref[...] + p1_ref[...]) + (p2_ref[...] + p3_ref[...])
    h = jax.nn.relu(_dot(nl, wa[...]) + _dot(agg, wb[...]) + b0[...])
    h = jax.nn.relu(_dot(h, w1[...]) + b1[...])
    h = _dot(h, w2[...]) + b2[...]
    o_ref[...] = nl + _ln(h, g[...], be[...], mm[...])


def _dec_body(x_ref, mm, w0, b0, w1, b1, w2, b2, o_ref):
    h = jax.nn.relu(_dot(x_ref[...], w0[...]) + b0[...])
    h = jax.nn.relu(_dot(h, w1[...]) + b1[...])
    o_ref[...] = _dot(h, w2[...]) + b2[...]


def _full_spec(shape):
    nd = len(shape)
    return pl.BlockSpec(shape, lambda i: (0,) * nd)


def _row_spec(bm, cols):
    return pl.BlockSpec((bm, cols), lambda i: (i, 0))


def _off_spec(bm, cols, off):
    return pl.BlockSpec((bm, cols), lambda i, _o=off: (i + _o, 0))


def _run_rows(body, x_list, w_list, out_cols, bm, n_out=1,
              x_offs=None, out_rows=None):
    rows = out_rows if out_rows is not None else x_list[0].shape[0]
    grid = rows // bm
    if x_offs is None:
        x_offs = [0] * len(x_list)
    in_specs = [_off_spec(bm, x.shape[1], o) for x, o in zip(x_list, x_offs)] \
        + [_full_spec(w.shape) for w in w_list]
    if n_out == 1:
        out_specs = _row_spec(bm, out_cols)
        out_shape = jax.ShapeDtypeStruct((rows, out_cols), jnp.float32)
    else:
        out_specs = [_row_spec(bm, out_cols) for _ in range(n_out)]
        out_shape = [jax.ShapeDtypeStruct((rows, out_cols), jnp.float32)
                     for _ in range(n_out)]
    return pl.pallas_call(
        body,
        grid=(grid,),
        in_specs=in_specs,
        out_specs=out_specs,
        out_shape=out_shape,
    )(*x_list, *w_list)


_MESH = plsc.VectorSubcoreMesh(core_axis_name="c", subcore_axis_name="s")


@functools.partial(
    pl.kernel,
    out_type=jax.ShapeDtypeStruct((N_EDGES, D), jnp.float32),
    mesh=_MESH,
    scratch_types=[
        [pltpu.VMEM((CHUNK,), jnp.int32) for _ in range(2)],
        [pltpu.VMEM((CHUNK,), jnp.int32) for _ in range(2)],
        [pltpu.VMEM((CHUNK, D), jnp.float32) for _ in range(2)],
        [pltpu.VMEM((CHUNK, D), jnp.float32) for _ in range(2)],
        [pltpu.SemaphoreType.DMA for _ in range(2)],
        [pltpu.SemaphoreType.DMA for _ in range(2)],
        [pltpu.SemaphoreType.DMA for _ in range(2)],
    ],
)
def _sc_gather(a_hbm, b_hbm, s_hbm, r_hbm, u1_hbm,
               si, ri, ra, rb, semi, semg, sems):
    wid = lax.axis_index("s") * NC + lax.axis_index("c")
    base = wid * EPW

    def issue_idx(c, t):
        off = pl.multiple_of(base + c * CHUNK, 8)
        pltpu.async_copy(s_hbm.at[pl.ds(off, CHUNK)], si[t], semi[t])
        pltpu.async_copy(r_hbm.at[pl.ds(off, CHUNK)], ri[t], semi[t])

    def wait_idx(t):
        pltpu.make_async_copy(s_hbm.at[pl.ds(0, CHUNK)], si[t], semi[t]).wait()
        pltpu.make_async_copy(r_hbm.at[pl.ds(0, CHUNK)], ri[t], semi[t]).wait()

    def issue_gather(t):
        pltpu.async_copy(a_hbm.at[si[t]], ra[t], semg[t])
        pltpu.async_copy(b_hbm.at[ri[t]], rb[t], semg[t])

    def wait_gather(t):
        pltpu.make_async_copy(u1_hbm.at[pl.ds(0, CHUNK)], ra[t], semg[t]).wait()
        pltpu.make_async_copy(u1_hbm.at[pl.ds(0, CHUNK)], rb[t], semg[t]).wait()

    def issue_store(c, t):
        off = pl.multiple_of(base + c * CHUNK, 8)
        pltpu.async_copy(ra[t], u1_hbm.at[pl.ds(off, CHUNK)], sems[t])

    def wait_store(t):
        pltpu.make_async_copy(ra[t], u1_hbm.at[pl.ds(0, CHUNK)], sems[t]).wait()

    def add(t):
        def add_row(e, c):
            for k in range(D // 16):
                sl = pl.ds(k * 16, 16)
                ra[t][e, sl] = ra[t][e, sl] + rb[t][e, sl]
            return c
        lax.fori_loop(0, CHUNK, add_row, 0)

    issue_idx(0, 0)
    issue_idx(1, 1)
    wait_idx(0)
    issue_gather(0)
    wait_idx(1)
    issue_gather(1)

    def body(k, carry):
        c = 2 * k
        wait_gather(0)
        issue_idx(c + 2, 0)
        add(0)
        issue_store(c, 0)
        wait_gather(1)
        issue_idx(c + 3, 1)
        wait_store(0)
        wait_idx(0)
        issue_gather(0)
        add(1)
        issue_store(c + 1, 1)
        wait_store(1)
        wait_idx(1)
        issue_gather(1)
        return carry

    lax.fori_loop(0, NCHUNK // 2 - 1, body, 0)

    wait_gather(0)
    add(0)
    issue_store(NCHUNK - 2, 0)
    wait_gather(1)
    add(1)
    issue_store(NCHUNK - 1, 1)
    wait_store(0)
    wait_store(1)


@functools.partial(
    pl.kernel,
    out_type=[jax.ShapeDtypeStruct((N_NODES, D), jnp.float32),
              jax.ShapeDtypeStruct((N_NODES, D), jnp.float32)],
    mesh=_MESH,
    scratch_types=[
        [pltpu.VMEM((CHUNK_S,), jnp.int32) for _ in range(2)],
        [pltpu.VMEM((CHUNK_S, D), jnp.float32) for _ in range(2)],
        pltpu.VMEM_SHARED((N_NODES, D), jnp.float32),
        [pltpu.SemaphoreType.DMA for _ in range(2)],
    ],
)
def _sc_scatter(ne_hbm, r_hbm, z_hbm, p0_hbm, p1_hbm, ri, rows, acc, seml):
    cid = lax.axis_index("c")
    sid = lax.axis_index("s")
    wid = sid * NC + cid
    base = wid * EPW_S
    tile_row = sid * ROWS_PER_TILE

    pltpu.sync_copy(z_hbm.at[pl.ds(tile_row, ROWS_PER_TILE)],
                    acc.at[pl.ds(tile_row, ROWS_PER_TILE)])

    @pl.when(sid == NS - 1)
    def _():
        pltpu.sync_copy(z_hbm.at[pl.ds(ROWS_PER_TILE * NS, ROWS_REM)],
                        acc.at[pl.ds(ROWS_PER_TILE * NS, ROWS_REM)])

    plsc.subcore_barrier()

    def issue_loads(c, t):
        off = pl.multiple_of(base + c * CHUNK_S, 8)
        pltpu.async_copy(r_hbm.at[pl.ds(off, CHUNK_S)], ri[t], seml[t])
        pltpu.async_copy(ne_hbm.at[pl.ds(off, CHUNK_S)], rows[t], seml[t])

    def wait_loads(t):
        pltpu.make_async_copy(r_hbm.at[pl.ds(0, CHUNK_S)], ri[t],
                              seml[t]).wait()
        pltpu.make_async_copy(ne_hbm.at[pl.ds(0, CHUNK_S)], rows[t],
                              seml[t]).wait()

    def issue_loads(c, t):
        off = pl.multiple_of(base + c * CHUNK_S, 8)
        di = pltpu.async_copy(r_hbm.at[pl.ds(off, CHUNK_S)], ri[t], seml[t])
        dr = pltpu.async_copy(ne_hbm.at[pl.ds(off, CHUNK_S)], rows[t],
                              seml[t])
        return di, dr

    def body(k, carry):
        c = 2 * k
        d0i, d0r = issue_loads(c, 0)
        d1i, d1r = issue_loads(c + 1, 1)
        d0i.wait()
        d0r.wait()
        pltpu.sync_copy(rows[0], acc.at[ri[0]], add=True)
        d1i.wait()
        d1r.wait()
        pltpu.sync_copy(rows[1], acc.at[ri[1]], add=True)
        return carry

    lax.fori_loop(0, NCHUNK_S // 2, body, 0)
    dli, dlr = issue_loads(NCHUNK_S - 1, 0)
    dli.wait()
    dlr.wait()
    pltpu.sync_copy(rows[0], acc.at[ri[0]], add=True)
    plsc.subcore_barrier()

    @pl.when(cid == 0)
    def _():
        pltpu.sync_copy(acc.at[pl.ds(tile_row, ROWS_PER_TILE)],
                        p0_hbm.at[pl.ds(tile_row, ROWS_PER_TILE)])

    @pl.when(cid == 1)
    def _():
        pltpu.sync_copy(acc.at[pl.ds(tile_row, ROWS_PER_TILE)],
                        p1_hbm.at[pl.ds(tile_row, ROWS_PER_TILE)])

    @pl.when((sid == NS - 1) & (cid == 0))
    def _():
        pltpu.sync_copy(acc.at[pl.ds(ROWS_PER_TILE * NS, ROWS_REM)],
                        p0_hbm.at[pl.ds(ROWS_PER_TILE * NS, ROWS_REM)])

    @pl.when((sid == NS - 1) & (cid == 1))
    def _():
        pltpu.sync_copy(acc.at[pl.ds(ROWS_PER_TILE * NS, ROWS_REM)],
                        p1_hbm.at[pl.ds(ROWS_PER_TILE * NS, ROWS_REM)])


def _pair(w):
    hi = w.astype(jnp.bfloat16)
    lo = (w - hi.astype(jnp.float32)).astype(jnp.bfloat16)
    return jnp.stack([jnp.concatenate([hi, lo], axis=0),
                      jnp.concatenate([lo, hi], axis=0)])


def _prep_mlp(p, pad_in=None):
    ws = [w for w in p['W']]
    bs = [b.reshape(1, -1) for b in p['b']]
    if pad_in is not None:
        ws[0] = jnp.pad(ws[0], ((0, pad_in - ws[0].shape[0]), (0, 0)))
    out = []
    for w, b in zip(ws, bs):
        out += [_pair(w), b]
    if 'ln_scale' in p:
        out += [p['ln_scale'].reshape(1, -1), p['ln_bias'].reshape(1, -1)]
    return out


def _edge_tables(blk):
    w1 = blk['edge']['W'][0]
    return (_pair(w1[:D]), blk['edge']['b'][0].reshape(1, -1),
            _pair(w1[D:2 * D]))


def kernel(node_features, edge_index, edge_features, params):
    senders = edge_index[0].astype(jnp.int32)
    receivers = edge_index[1].astype(jnp.int32)
    blocks = params['blocks']

    enc_n = _prep_mlp(params['enc_node'])
    enc_e = _prep_mlp(params['enc_edge'], pad_in=8)
    ef8 = jnp.pad(edge_features, ((0, 0), (0, 4)))

    mm = jnp.full((2 * D, D), 1.0 / D, jnp.bfloat16)
    ws0, bs0, wr0 = _edge_tables(blocks[0])
    nl, ta, tb = _run_rows(_enc_node_body, [node_features],
                           [mm] + enc_n + [ws0, bs0, wr0], D, 1000, n_out=3)
    BE = 2000
    HB = EH // BE
    el0 = _run_rows(_enc_edge_body, [ef8], [mm] + enc_e, D, BE, out_rows=EH)
    el1 = _run_rows(_enc_edge_body, [ef8], [mm] + enc_e, D, BE, out_rows=EH,
                    x_offs=[HB])

    r0 = receivers[:EH]
    r1 = receivers[EH:]
    zeros_tab = jnp.zeros((N_NODES, D), jnp.float32)

    for s, blk in enumerate(blocks):
        pe = blk['edge']
        w1 = pe['W'][0]
        ew = [mm, _pair(w1[2 * D:]),
              _pair(pe['W'][1]), pe['b'][1].reshape(1, -1),
              _pair(pe['W'][2]), pe['b'][2].reshape(1, -1),
              pe['ln_scale'].reshape(1, -1), pe['ln_bias'].reshape(1, -1)]
        pn = blk['node']
        nw1 = pn['W'][0]
        nwl = [mm, _pair(nw1[:D]), _pair(nw1[D:]),
               pn['b'][0].reshape(1, -1),
               _pair(pn['W'][1]), pn['b'][1].reshape(1, -1),
               _pair(pn['W'][2]), pn['b'][2].reshape(1, -1),
               pn['ln_scale'].reshape(1, -1), pn['ln_bias'].reshape(1, -1)]

        u1 = _sc_gather(ta, tb, senders, receivers)
        el0 = _run_rows(_edge_step_body, [u1, el0], ew, D, BE, out_rows=EH)
        p0, p1 = _sc_scatter(el0, r0, zeros_tab)
        el1 = _run_rows(_edge_step_body, [u1, el1], ew, D, BE, out_rows=EH,
                        x_offs=[HB, 0])
        p2, p3 = _sc_scatter(el1, r1, zeros_tab)
        if s + 1 < len(blocks):
            wsn, bsn, wrn = _edge_tables(blocks[s + 1])
            nl, ta, tb = _run_rows(_node_step_body, [nl, p0, p1, p2, p3],
                                   nwl + [wsn, bsn, wrn], D, 1000, n_out=3)
        else:
            nl = _run_rows(_node_last_body, [nl, p0, p1, p2, p3],
                           nwl, D, 1000)

    pd = params['dec']
    dec = [mm,
           _pair(pd['W'][0]), pd['b'][0].reshape(1, -1),
           _pair(pd['W'][1]), pd['b'][1].reshape(1, -1),
           _pair(jnp.pad(pd['W'][2], ((0, 0), (0, 5)))),
           jnp.pad(pd['b'][2].reshape(1, -1), ((0, 0), (0, 5)))]
    out8 = _run_rows(_dec_body, [nl], dec, 8, 1000)
    return out8[:, :3]


# speedup vs baseline: 3.7502x; 1.0867x over previous
import functools

import jax
import jax.numpy as jnp
from jax import lax
from jax.experimental import pallas as pl
from jax.experimental.pallas import tpu as pltpu
from jax.experimental.pallas import tpu_sc as plsc

N_NODES = 10000
N_EDGES = 320000
D = 128

NC = 2
NS = 16
NW = NC * NS
EPW = N_EDGES // NW
EH = N_EDGES // 2
EPW_G = EH // NW
CHUNK = 200
NCHUNK = EPW_G // CHUNK
CHUNK_S = 40
NCHUNK_S = EPW_G // CHUNK_S
ROWS_PER_TILE = 624
ROWS_REM = N_NODES - ROWS_PER_TILE * NS


def _split(x):
    xh = x.astype(jnp.bfloat16)
    xl = (x - xh.astype(jnp.float32)).astype(jnp.bfloat16)
    return xh, xl


def _wide(x):
    xh, xl = _split(x)
    return jnp.concatenate([xh, xl], axis=-1)


def _dot(x, wp):
    xc = _wide(x)
    return (jnp.dot(xc, wp[0], preferred_element_type=jnp.float32)
            + jnp.dot(xc, wp[1], preferred_element_type=jnp.float32))


def _ln(x, g, b, mm):
    mu = jnp.dot(_wide(x), mm, preferred_element_type=jnp.float32)
    xc = x - mu
    var = jnp.dot(_wide(xc * xc), mm, preferred_element_type=jnp.float32)
    return xc / jnp.sqrt(var + 1e-5) * g + b


def _enc_node_body(x_ref, mm, w0, b0, w1, b1, w2, b2, g, be, ws, bs, wr,
                   o_ref, a_ref, t_ref):
    h = jax.nn.relu(_dot(x_ref[...], w0[...]) + b0[...])
    h = jax.nn.relu(_dot(h, w1[...]) + b1[...])
    h = _dot(h, w2[...]) + b2[...]
    nl = _ln(h, g[...], be[...], mm[...])
    o_ref[...] = nl
    a_ref[...] = _dot(nl, ws[...]) + bs[...]
    t_ref[...] = _dot(nl, wr[...])


def _enc_edge_body(x_ref, mm, w0, b0, w1, b1, w2, b2, g, be, o_ref):
    h = jax.nn.relu(_dot(x_ref[...], w0[...]) + b0[...])
    h = jax.nn.relu(_dot(h, w1[...]) + b1[...])
    h = _dot(h, w2[...]) + b2[...]
    o_ref[...] = _ln(h, g[...], be[...], mm[...])


def _edge_step_body(u1_ref, el_ref, mm, we, w1, bb1, w2, b2, g, be, o_ref):
    el = el_ref[...]
    h = jax.nn.relu(u1_ref[...] + _dot(el, we[...]))
    h = jax.nn.relu(_dot(h, w1[...]) + bb1[...])
    h = _dot(h, w2[...]) + b2[...]
    o_ref[...] = el + _ln(h, g[...], be[...], mm[...])


def _node_step_body(nl_ref, p0_ref, p1_ref, p2_ref, p3_ref, mm,
                    wa, wb, b0, w1, b1, w2, b2,
                    g, be, ws, bs, wr, o_ref, a_ref, t_ref):
    nl = nl_ref[...]
    agg = (p0_ref[...] + p1_ref[...]) + (p2_ref[...] + p3_ref[...])
    h = jax.nn.relu(_dot(nl, wa[...]) + _dot(agg, wb[...]) + b0[...])
    h = jax.nn.relu(_dot(h, w1[...]) + b1[...])
    h = _dot(h, w2[...]) + b2[...]
    nl = nl + _ln(h, g[...], be[...], mm[...])
    o_ref[...] = nl
    a_ref[...] = _dot(nl, ws[...]) + bs[...]
    t_ref[...] = _dot(nl, wr[...])


def _node_last_body(nl_ref, p0_ref, p1_ref, p2_ref, p3_ref, mm,
                    wa, wb, b0, w1, b1, w2, b2, g, be, o_ref):
    nl = nl_ref[...]
    agg = (p0_ref[...] + p1_ref[...]) + (p2_ref[...] + p3_ref[...])
    h = jax.nn.relu(_dot(nl, wa[...]) + _dot(agg, wb[...]) + b0[...])
    h = jax.nn.relu(_dot(h, w1[...]) + b1[...])
    h = _dot(h, w2[...]) + b2[...]
    o_ref[...] = nl + _ln(h, g[...], be[...], mm[...])


def _dec_body(x_ref, mm, w0, b0, w1, b1, w2, b2, o_ref):
    h = jax.nn.relu(_dot(x_ref[...], w0[...]) + b0[...])
    h = jax.nn.relu(_dot(h, w1[...]) + b1[...])
    o_ref[...] = _dot(h, w2[...]) + b2[...]


def _full_spec(shape):
    nd = len(shape)
    return pl.BlockSpec(shape, lambda i: (0,) * nd)


def _row_spec(bm, cols):
    return pl.BlockSpec((bm, cols), lambda i: (i, 0))


def _off_spec(bm, cols, off):
    return pl.BlockSpec((bm, cols), lambda i, _o=off: (i + _o, 0))


def _run_rows(body, x_list, w_list, out_cols, bm, n_out=1,
              x_offs=None, out_rows=None):
    rows = out_rows if out_rows is not None else x_list[0].shape[0]
    grid = rows // bm
    if x_offs is None:
        x_offs = [0] * len(x_list)
    in_specs = [_off_spec(bm, x.shape[1], o) for x, o in zip(x_list, x_offs)] \
        + [_full_spec(w.shape) for w in w_list]
    if n_out == 1:
        out_specs = _row_spec(bm, out_cols)
        out_shape = jax.ShapeDtypeStruct((rows, out_cols), jnp.float32)
    else:
        out_specs = [_row_spec(bm, out_cols) for _ in range(n_out)]
        out_shape = [jax.ShapeDtypeStruct((rows, out_cols), jnp.float32)
                     for _ in range(n_out)]
    return pl.pallas_call(
        body,
        grid=(grid,),
        in_specs=in_specs,
        out_specs=out_specs,
        out_shape=out_shape,
    )(*x_list, *w_list)


_MESH = plsc.VectorSubcoreMesh(core_axis_name="c", subcore_axis_name="s")


@functools.partial(
    pl.kernel,
    out_type=jax.ShapeDtypeStruct((EH, D), jnp.float32),
    mesh=_MESH,
    scratch_types=[
        [pltpu.VMEM((CHUNK,), jnp.int32) for _ in range(2)],
        [pltpu.VMEM((CHUNK,), jnp.int32) for _ in range(2)],
        [pltpu.VMEM((CHUNK, D), jnp.float32) for _ in range(2)],
        [pltpu.VMEM((CHUNK, D), jnp.float32) for _ in range(2)],
        [pltpu.SemaphoreType.DMA for _ in range(2)],
        [pltpu.SemaphoreType.DMA for _ in range(2)],
        [pltpu.SemaphoreType.DMA for _ in range(2)],
    ],
)
def _sc_gather(a_hbm, b_hbm, s_hbm, r_hbm, u1_hbm,
               si, ri, ra, rb, semi, semg, sems):
    wid = lax.axis_index("s") * NC + lax.axis_index("c")
    base = wid * EPW_G

    def issue_idx(c, t):
        off = pl.multiple_of(base + c * CHUNK, 8)
        pltpu.async_copy(s_hbm.at[pl.ds(off, CHUNK)], si[t], semi[t])
        pltpu.async_copy(r_hbm.at[pl.ds(off, CHUNK)], ri[t], semi[t])

    def wait_idx(t):
        pltpu.make_async_copy(s_hbm.at[pl.ds(0, CHUNK)], si[t], semi[t]).wait()
        pltpu.make_async_copy(r_hbm.at[pl.ds(0, CHUNK)], ri[t], semi[t]).wait()

    def issue_gather(t):
        pltpu.async_copy(a_hbm.at[si[t]], ra[t], semg[t])
        pltpu.async_copy(b_hbm.at[ri[t]], rb[t], semg[t])

    def wait_gather(t):
        pltpu.make_async_copy(u1_hbm.at[pl.ds(0, CHUNK)], ra[t], semg[t]).wait()
        pltpu.make_async_copy(u1_hbm.at[pl.ds(0, CHUNK)], rb[t], semg[t]).wait()

    def issue_store(c, t):
        off = pl.multiple_of(base + c * CHUNK, 8)
        pltpu.async_copy(ra[t], u1_hbm.at[pl.ds(off, CHUNK)], sems[t])

    def wait_store(t):
        pltpu.make_async_copy(ra[t], u1_hbm.at[pl.ds(0, CHUNK)], sems[t]).wait()

    def add(t):
        def add_row(e, c):
            for k in range(D // 16):
                sl = pl.ds(k * 16, 16)
                ra[t][e, sl] = ra[t][e, sl] + rb[t][e, sl]
            return c
        lax.fori_loop(0, CHUNK, add_row, 0)

    issue_idx(0, 0)
    issue_idx(1, 1)
    wait_idx(0)
    issue_gather(0)
    wait_idx(1)
    issue_gather(1)

    def body(k, carry):
        c = 2 * k
        wait_gather(0)
        issue_idx(c + 2, 0)
        add(0)
        issue_store(c, 0)
        wait_gather(1)
        issue_idx(c + 3, 1)
        wait_store(0)
        wait_idx(0)
        issue_gather(0)
        add(1)
        issue_store(c + 1, 1)
        wait_store(1)
        wait_idx(1)
        issue_gather(1)
        return carry

    lax.fori_loop(0, (NCHUNK - 3) // 2, body, 0)

    wait_gather(0)
    add(0)
    issue_store(NCHUNK - 3, 0)
    wait_store(0)
    issue_idx(NCHUNK - 1, 0)
    wait_idx(0)
    issue_gather(0)
    wait_gather(1)
    add(1)
    issue_store(NCHUNK - 2, 1)
    wait_gather(0)
    add(0)
    issue_store(NCHUNK - 1, 0)
    wait_store(0)
    wait_store(1)


@functools.partial(
    pl.kernel,
    out_type=[jax.ShapeDtypeStruct((N_NODES, D), jnp.float32),
              jax.ShapeDtypeStruct((N_NODES, D), jnp.float32)],
    mesh=_MESH,
    scratch_types=[
        [pltpu.VMEM((CHUNK_S,), jnp.int32) for _ in range(2)],
        [pltpu.VMEM((CHUNK_S, D), jnp.float32) for _ in range(2)],
        pltpu.VMEM_SHARED((N_NODES, D), jnp.float32),
        [pltpu.SemaphoreType.DMA for _ in range(2)],
    ],
)
def _sc_scatter(ne_hbm, r_hbm, z_hbm, p0_hbm, p1_hbm, ri, rows, acc, seml):
    cid = lax.axis_index("c")
    sid = lax.axis_index("s")
    wid = sid * NC + cid
    base = wid * EPW_G
    tile_row = sid * ROWS_PER_TILE

    pltpu.sync_copy(z_hbm.at[pl.ds(tile_row, ROWS_PER_TILE)],
                    acc.at[pl.ds(tile_row, ROWS_PER_TILE)])

    @pl.when(sid == NS - 1)
    def _():
        pltpu.sync_copy(z_hbm.at[pl.ds(ROWS_PER_TILE * NS, ROWS_REM)],
                        acc.at[pl.ds(ROWS_PER_TILE * NS, ROWS_REM)])

    plsc.subcore_barrier()

    def issue_loads(c, t):
        off = pl.multiple_of(base + c * CHUNK_S, 8)
        pltpu.async_copy(r_hbm.at[pl.ds(off, CHUNK_S)], ri[t], seml[t])
        pltpu.async_copy(ne_hbm.at[pl.ds(off, CHUNK_S)], rows[t], seml[t])

    def wait_loads(t):
        pltpu.make_async_copy(r_hbm.at[pl.ds(0, CHUNK_S)], ri[t],
                              seml[t]).wait()
        pltpu.make_async_copy(ne_hbm.at[pl.ds(0, CHUNK_S)], rows[t],
                              seml[t]).wait()

    def issue_loads(c, t):
        off = pl.multiple_of(base + c * CHUNK_S, 8)
        di = pltpu.async_copy(r_hbm.at[pl.ds(off, CHUNK_S)], ri[t], seml[t])
        dr = pltpu.async_copy(ne_hbm.at[pl.ds(off, CHUNK_S)], rows[t],
                              seml[t])
        return di, dr

    def body(k, carry):
        c = 2 * k
        d0i, d0r = issue_loads(c, 0)
        d1i, d1r = issue_loads(c + 1, 1)
        d0i.wait()
        d0r.wait()
        pltpu.sync_copy(rows[0], acc.at[ri[0]], add=True)
        d1i.wait()
        d1r.wait()
        pltpu.sync_copy(rows[1], acc.at[ri[1]], add=True)
        return carry

    lax.fori_loop(0, NCHUNK_S // 2, body, 0)
    dli, dlr = issue_loads(NCHUNK_S - 1, 0)
    dli.wait()
    dlr.wait()
    pltpu.sync_copy(rows[0], acc.at[ri[0]], add=True)
    plsc.subcore_barrier()

    @pl.when(cid == 0)
    def _():
        pltpu.sync_copy(acc.at[pl.ds(tile_row, ROWS_PER_TILE)],
                        p0_hbm.at[pl.ds(tile_row, ROWS_PER_TILE)])

    @pl.when(cid == 1)
    def _():
        pltpu.sync_copy(acc.at[pl.ds(tile_row, ROWS_PER_TILE)],
                        p1_hbm.at[pl.ds(tile_row, ROWS_PER_TILE)])

    @pl.when((sid == NS - 1) & (cid == 0))
    def _():
        pltpu.sync_copy(acc.at[pl.ds(ROWS_PER_TILE * NS, ROWS_REM)],
                        p0_hbm.at[pl.ds(ROWS_PER_TILE * NS, ROWS_REM)])

    @pl.when((sid == NS - 1) & (cid == 1))
    def _():
        pltpu.sync_copy(acc.at[pl.ds(ROWS_PER_TILE * NS, ROWS_REM)],
                        p1_hbm.at[pl.ds(ROWS_PER_TILE * NS, ROWS_REM)])


def _pair(w):
    hi = w.astype(jnp.bfloat16)
    lo = (w - hi.astype(jnp.float32)).astype(jnp.bfloat16)
    return jnp.stack([jnp.concatenate([hi, lo], axis=0),
                      jnp.concatenate([lo, hi], axis=0)])


def _prep_mlp(p, pad_in=None):
    ws = [w for w in p['W']]
    bs = [b.reshape(1, -1) for b in p['b']]
    if pad_in is not None:
        ws[0] = jnp.pad(ws[0], ((0, pad_in - ws[0].shape[0]), (0, 0)))
    out = []
    for w, b in zip(ws, bs):
        out += [_pair(w), b]
    if 'ln_scale' in p:
        out += [p['ln_scale'].reshape(1, -1), p['ln_bias'].reshape(1, -1)]
    return out


def _edge_tables(blk):
    w1 = blk['edge']['W'][0]
    return (_pair(w1[:D]), blk['edge']['b'][0].reshape(1, -1),
            _pair(w1[D:2 * D]))


def kernel(node_features, edge_index, edge_features, params):
    senders = edge_index[0].astype(jnp.int32)
    receivers = edge_index[1].astype(jnp.int32)
    blocks = params['blocks']

    enc_n = _prep_mlp(params['enc_node'])
    enc_e = _prep_mlp(params['enc_edge'], pad_in=8)
    ef8 = jnp.pad(edge_features, ((0, 0), (0, 4)))

    mm = jnp.full((2 * D, D), 1.0 / D, jnp.bfloat16)
    ws0, bs0, wr0 = _edge_tables(blocks[0])
    nl, ta, tb = _run_rows(_enc_node_body, [node_features],
                           [mm] + enc_n + [ws0, bs0, wr0], D, 1000, n_out=3)
    BE = 2000
    HB = EH // BE
    el0 = _run_rows(_enc_edge_body, [ef8], [mm] + enc_e, D, BE, out_rows=EH)
    el1 = _run_rows(_enc_edge_body, [ef8], [mm] + enc_e, D, BE, out_rows=EH,
                    x_offs=[HB])

    s0 = senders[:EH]
    s1 = senders[EH:]
    r0 = receivers[:EH]
    r1 = receivers[EH:]
    zeros_tab = jnp.zeros((N_NODES, D), jnp.float32)

    for s, blk in enumerate(blocks):
        pe = blk['edge']
        w1 = pe['W'][0]
        ew = [mm, _pair(w1[2 * D:]),
              _pair(pe['W'][1]), pe['b'][1].reshape(1, -1),
              _pair(pe['W'][2]), pe['b'][2].reshape(1, -1),
              pe['ln_scale'].reshape(1, -1), pe['ln_bias'].reshape(1, -1)]
        pn = blk['node']
        nw1 = pn['W'][0]
        nwl = [mm, _pair(nw1[:D]), _pair(nw1[D:]),
               pn['b'][0].reshape(1, -1),
               _pair(pn['W'][1]), pn['b'][1].reshape(1, -1),
               _pair(pn['W'][2]), pn['b'][2].reshape(1, -1),
               pn['ln_scale'].reshape(1, -1), pn['ln_bias'].reshape(1, -1)]

        u10 = _sc_gather(ta, tb, s0, r0)
        u11 = _sc_gather(ta, tb, s1, r1)
        el0 = _run_rows(_edge_step_body, [u10, el0], ew, D, BE, out_rows=EH)
        p0, p1 = _sc_scatter(el0, r0, zeros_tab)
        el1 = _run_rows(_edge_step_body, [u11, el1], ew, D, BE, out_rows=EH)
        p2, p3 = _sc_scatter(el1, r1, zeros_tab)
        if s + 1 < len(blocks):
            wsn, bsn, wrn = _edge_tables(blocks[s + 1])
            nl, ta, tb = _run_rows(_node_step_body, [nl, p0, p1, p2, p3],
                                   nwl + [wsn, bsn, wrn], D, 1000, n_out=3)
        else:
            nl = _run_rows(_node_last_body, [nl, p0, p1, p2, p3],
                           nwl, D, 1000)

    pd = params['dec']
    dec = [mm,
           _pair(pd['W'][0]), pd['b'][0].reshape(1, -1),
           _pair(pd['W'][1]), pd['b'][1].reshape(1, -1),
           _pair(jnp.pad(pd['W'][2], ((0, 0), (0, 5)))),
           jnp.pad(pd['b'][2].reshape(1, -1), ((0, 0), (0, 5)))]
    out8 = _run_rows(_dec_body, [nl], dec, 8, 1000)
    return out8[:, :3]
